# baseline (speedup 1.0000x reference)
import functools

import jax
import jax.numpy as jnp
import numpy as np
from jax import lax
from jax.experimental import pallas as pl
from jax.experimental.pallas import tpu as pltpu
from jax.experimental.pallas import tpu_sc as plsc

N = 10000
E = 160000
EMB = 128
HID = 512
L = 5
G = 256
TASKS = 128
PE = 10
ATOM_DIMS = [119, 5, 12, 12, 10, 6, 6, 2, 2]
BOND_DIMS = [5, 6, 2]
ATOM_OFF = np.concatenate([[0], np.cumsum(ATOM_DIMS)[:-1]]).astype(np.int32)
BOND_OFF = np.concatenate([[0], np.cumsum(BOND_DIMS)[:-1]]).astype(np.int32)
A_PAD = 256
B_PAD = 16

NPAD = 10240
CHUNK = 80
NCHUNK = E // CHUNK
NWORK = 32
CPW = (NCHUNK + NWORK - 1) // NWORK

NB = 2000
EB = 2000


def _embed_h_body(h_ref, pos_ref, sign_ref, at_ref, pw_ref, pb_ref, out_ref):
    hb = h_ref[...]
    acc = jnp.zeros((NB, A_PAD), jnp.float32)
    for i in range(9):
        col = hb[:, i:i + 1] + np.int32(ATOM_OFF[i])
        ids = lax.broadcasted_iota(jnp.int32, (NB, A_PAD), 1)
        acc = acc + (ids == col).astype(jnp.float32)
    hx = jnp.dot(acc, at_ref[...], preferred_element_type=jnp.float32)
    pos = pos_ref[...] * sign_ref[0:1, :]
    hx = hx + jnp.dot(pos, pw_ref[...], preferred_element_type=jnp.float32)
    out_ref[...] = hx + pb_ref[0:1, :]


def _embed_e_body(ef_ref, bt_ref, wc_ref, bc_ref, ex_ref, ce_ref):
    eb = ef_ref[...]
    acc = jnp.zeros((EB, B_PAD), jnp.float32)
    for i in range(3):
        col = eb[:, i:i + 1] + np.int32(BOND_OFF[i])
        ids = lax.broadcasted_iota(jnp.int32, (EB, B_PAD), 1)
        acc = acc + (ids == col).astype(jnp.float32)
    ex = jnp.dot(acc, bt_ref[...], preferred_element_type=jnp.float32)
    ex_ref[...] = ex
    ce_ref[...] = jnp.dot(ex, wc_ref[...], preferred_element_type=jnp.float32) \
        + bc_ref[0:1, :]


def _nodemm_body(hx_ref, b_ref, vn_ref, w_ref, bias_ref,
                 heff_ref, a_ref, bd_ref, e_ref):
    batch = b_ref[...]
    ids = lax.broadcasted_iota(jnp.int32, (NB, G), 1)
    mask = (ids == batch).astype(jnp.float32)
    heff = hx_ref[...] + jnp.dot(mask, vn_ref[...],
                                 preferred_element_type=jnp.float32)
    p = jnp.dot(heff, w_ref[...], preferred_element_type=jnp.float32) \
        + bias_ref[0:1, :]
    heff_ref[...] = heff
    a_ref[...] = p[:, 0:EMB]
    bd_ref[...] = p[:, EMB:3 * EMB]
    e_ref[...] = p[:, 3 * EMB:4 * EMB]


def _nodeup1_body(a_ref, n0_ref, n1_ref, d0_ref, d1_ref, hn_ref, st_ref):
    num = n0_ref[...] + n1_ref[...]
    den = d0_ref[...] + d1_ref[...] + 1e-6
    hnew = a_ref[...] + num / den
    hn_ref[...] = hnew
    s = jnp.sum(hnew, axis=0, keepdims=True)
    sq = jnp.sum(hnew * hnew, axis=0, keepdims=True)
    part = jnp.concatenate(
        [s, sq, jnp.zeros((6, EMB), jnp.float32)], axis=0)

    @pl.when(pl.program_id(0) == 0)
    def _init():
        st_ref[...] = jnp.zeros_like(st_ref)

    st_ref[...] += part


def _nodeup2_body(hn_ref, heff_ref, st_ref, g_ref, bb_ref, b_ref,
                  hx_ref, ps_ref, pc_ref):
    s = st_ref[...]
    mu = s[0:1, :] / N
    var = s[1:2, :] / N - mu * mu
    hnew = (hn_ref[...] - mu) / jnp.sqrt(var + 1e-5) * g_ref[0:1, :] \
        + bb_ref[0:1, :]
    hx = heff_ref[...] + jnp.maximum(hnew, 0.0)
    hx_ref[...] = hx
    batch = b_ref[...]
    bcol = lax.broadcast_in_dim(batch[:, 0], (G, NB), (1,))
    mask = (bcol == lax.broadcasted_iota(jnp.int32, (G, NB), 0)) \
        .astype(jnp.float32)
    part = jnp.dot(mask, hx, preferred_element_type=jnp.float32)
    cnt = jnp.sum(mask, axis=1, keepdims=True)

    @pl.when(pl.program_id(0) == 0)
    def _init():
        ps_ref[...] = jnp.zeros_like(ps_ref)
        pc_ref[...] = jnp.zeros_like(pc_ref)

    ps_ref[...] += part
    pc_ref[...] += jnp.broadcast_to(cnt, (G, EMB))


def _vn_body(vn_ref, ps_ref, pc_ref, g_ref, b_ref, w1_ref, b1_ref, w2_ref,
             b2_ref, out_ref):
    pool = ps_ref[...] / jnp.maximum(pc_ref[...], 1.0)
    vn = vn_ref[...] + pool
    mu = jnp.mean(vn, axis=0, keepdims=True)
    var = jnp.mean(vn * vn, axis=0, keepdims=True) - mu * mu
    t = (vn - mu) / jnp.sqrt(var + 1e-5) * g_ref[0:1, :] + b_ref[0:1, :]
    t = jnp.maximum(jnp.dot(t, w1_ref[...],
                            preferred_element_type=jnp.float32)
                    + b1_ref[0:1, :], 0.0)
    out_ref[...] = jnp.dot(t, w2_ref[...],
                           preferred_element_type=jnp.float32) + b2_ref[0:1, :]


def _ebn(en, esum_ref, esq_ref, g_ref, b_ref):
    mu = jnp.sum(esum_ref[...], axis=0, keepdims=True) / E
    var = jnp.sum(esq_ref[...], axis=0, keepdims=True) / E - mu * mu
    return jnp.maximum((en - mu) / jnp.sqrt(var + 1e-5) * g_ref[0:1, :]
                       + b_ref[0:1, :], 0.0)


def _edgeup_body(en_ref, ex_ref, esum_ref, esq_ref, g_ref, b_ref,
                 wc_ref, bc_ref, exo_ref, ce_ref):
    ex = ex_ref[...] + _ebn(en_ref[...], esum_ref, esq_ref, g_ref, b_ref)
    exo_ref[...] = ex
    ce_ref[...] = jnp.dot(ex, wc_ref[...],
                          preferred_element_type=jnp.float32) + bc_ref[0:1, :]


def _edgeup_final_body(en_ref, ex_ref, esum_ref, esq_ref, g_ref, b_ref,
                       eb_ref, ps_ref, pc_ref):
    ex = ex_ref[...] + _ebn(en_ref[...], esum_ref, esq_ref, g_ref, b_ref)
    ebatch = eb_ref[...]
    bcol = lax.broadcast_in_dim(ebatch[:, 0], (G, EB), (1,))
    mask = (bcol == lax.broadcasted_iota(jnp.int32, (G, EB), 0)) \
        .astype(jnp.float32)
    part = jnp.dot(mask, ex, preferred_element_type=jnp.float32)
    cnt = jnp.sum(mask, axis=1, keepdims=True)

    @pl.when(pl.program_id(0) == 0)
    def _init():
        ps_ref[...] = jnp.zeros_like(ps_ref)
        pc_ref[...] = jnp.zeros_like(pc_ref)

    ps_ref[...] += part
    pc_ref[...] += jnp.broadcast_to(cnt, (G, EMB))


def _pred_body(nps_ref, npc_ref, es_ref, ec_ref, w1_ref, b1_ref, w2_ref,
               b2_ref, out_ref):
    node_pool = nps_ref[...] / jnp.maximum(npc_ref[...], 1.0)
    epool = es_ref[...] / jnp.maximum(ec_ref[...], 1.0)
    hg = jnp.concatenate([node_pool, epool], axis=-1)
    t = jnp.maximum(jnp.dot(hg, w1_ref[...],
                            preferred_element_type=jnp.float32)
                    + b1_ref[0:1, :], 0.0)
    out_ref[...] = jnp.dot(t, w2_ref[...],
                           preferred_element_type=jnp.float32) + b2_ref[0:1, :]


def _sc_zero_buf(buf):
    def zrow(r, _):
        for j in range(EMB // 16):
            buf[r, pl.ds(j * 16, 16)] = jnp.zeros((16,), jnp.float32)
        return 0
    lax.fori_loop(0, CHUNK, zrow, 0)


def _sc_zero_acc(acc, zbuf, sid):
    _sc_zero_buf(zbuf)
    for t in range(640 // CHUNK):
        pltpu.sync_copy(zbuf, acc.at[pl.ds(sid * 640 + t * CHUNK, CHUNK)])


def _sc_copy_out(acc, cid, sid, out0, out1):
    @pl.when(cid == 0)
    def _():
        pltpu.sync_copy(acc.at[pl.ds(sid * 640, 640)],
                        out0.at[pl.ds(sid * 640, 640)])

    @pl.when(cid == 1)
    def _():
        pltpu.sync_copy(acc.at[pl.ds(sid * 640, 640)],
                        out1.at[pl.ds(sid * 640, 640)])


def _edge_a_body(want_ebatch,
                 bd_hbm, eh_hbm, ce_hbm, src_hbm, dst_hbm, batch_hbm,
                 enew_hbm, num0_hbm, num1_hbm, esum_hbm, esq_hbm, ebatch_hbm,
                 src_v, dst_v, bdv, ev, cv, btv, stats_v, acc, sem):
    cid = lax.axis_index("c")
    sid = lax.axis_index("s")
    wid = sid * 2 + cid

    _sc_zero_acc(acc, cv, sid)
    for r in range(2):
        for j in range(EMB // 16):
            stats_v[r, pl.ds(j * 16, 16)] = jnp.zeros((16,), jnp.float32)
    plsc.subcore_barrier()

    def chunk_body(i, _):
        chunk = wid + NWORK * i

        @pl.when(chunk < NCHUNK)
        def _run():
            base = chunk * CHUNK
            pltpu.sync_copy(src_hbm.at[pl.ds(base, CHUNK)], src_v)
            pltpu.sync_copy(dst_hbm.at[pl.ds(base, CHUNK)], dst_v)
            c1 = pltpu.async_copy(bd_hbm.at[src_v], bdv, sem)
            c2 = pltpu.async_copy(eh_hbm.at[dst_v], ev, sem)
            pltpu.sync_copy(ce_hbm.at[pl.ds(base, CHUNK)], cv)
            c1.wait()
            c2.wait()

            def row(r, _):
                for j in range(EMB // 16):
                    sl = pl.ds(j * 16, 16)
                    e = cv[r, sl] + bdv[r, pl.ds(EMB + j * 16, 16)] \
                        + ev[r, sl]
                    cv[r, sl] = e
                    sg = 1.0 / (1.0 + jnp.exp(-e))
                    ev[r, sl] = sg * bdv[r, sl]
                    stats_v[0, sl] = stats_v[0, sl] + e
                    stats_v[1, sl] = stats_v[1, sl] + e * e
                return 0

            lax.fori_loop(0, CHUNK, row, 0)
            pltpu.sync_copy(cv, enew_hbm.at[pl.ds(base, CHUNK)])
            pltpu.sync_copy(ev, acc.at[dst_v], add=True)
            if want_ebatch:
                pltpu.async_copy(batch_hbm.at[dst_v], btv, sem).wait()
                pltpu.sync_copy(btv, ebatch_hbm.at[pl.ds(base, CHUNK)])
        return 0

    lax.fori_loop(0, CPW, chunk_body, 0)
    pltpu.sync_copy(stats_v.at[0], esum_hbm.at[wid])
    pltpu.sync_copy(stats_v.at[1], esq_hbm.at[wid])
    plsc.subcore_barrier()
    _sc_copy_out(acc, cid, sid, num0_hbm, num1_hbm)


def _edge_b_body(enew_hbm, dst_hbm, den0_hbm, den1_hbm,
                 dst_v, cv, acc, sem):
    cid = lax.axis_index("c")
    sid = lax.axis_index("s")
    wid = sid * 2 + cid

    _sc_zero_acc(acc, cv, sid)
    plsc.subcore_barrier()

    def chunk_body(i, _):
        chunk = wid + NWORK * i

        @pl.when(chunk < NCHUNK)
        def _run():
            base = chunk * CHUNK
            pltpu.sync_copy(dst_hbm.at[pl.ds(base, CHUNK)], dst_v)
            pltpu.sync_copy(enew_hbm.at[pl.ds(base, CHUNK)], cv)

            def row(r, _):
                for j in range(EMB // 16):
                    sl = pl.ds(j * 16, 16)
                    cv[r, sl] = 1.0 / (1.0 + jnp.exp(-cv[r, sl]))
                return 0

            lax.fori_loop(0, CHUNK, row, 0)
            pltpu.sync_copy(cv, acc.at[dst_v], add=True)
        return 0

    lax.fori_loop(0, CPW, chunk_body, 0)
    plsc.subcore_barrier()
    _sc_copy_out(acc, cid, sid, den0_hbm, den1_hbm)


def _make_edge_a(want_ebatch):
    mesh = plsc.VectorSubcoreMesh(core_axis_name="c", subcore_axis_name="s")
    return pl.kernel(
        functools.partial(_edge_a_body, want_ebatch),
        out_type=(
            jax.ShapeDtypeStruct((E, EMB), jnp.float32),
            jax.ShapeDtypeStruct((NPAD, EMB), jnp.float32),
            jax.ShapeDtypeStruct((NPAD, EMB), jnp.float32),
            jax.ShapeDtypeStruct((NWORK, EMB), jnp.float32),
            jax.ShapeDtypeStruct((NWORK, EMB), jnp.float32),
            jax.ShapeDtypeStruct((E,), jnp.int32),
        ),
        mesh=mesh,
        scratch_types=[
            pltpu.VMEM((CHUNK,), jnp.int32),
            pltpu.VMEM((CHUNK,), jnp.int32),
            pltpu.VMEM((CHUNK, 2 * EMB), jnp.float32),
            pltpu.VMEM((CHUNK, EMB), jnp.float32),
            pltpu.VMEM((CHUNK, EMB), jnp.float32),
            pltpu.VMEM((CHUNK,), jnp.int32),
            pltpu.VMEM((2, EMB), jnp.float32),
            pltpu.VMEM_SHARED((NPAD, EMB), jnp.float32),
            pltpu.SemaphoreType.DMA,
        ],
    )


def _make_edge_b():
    mesh = plsc.VectorSubcoreMesh(core_axis_name="c", subcore_axis_name="s")
    return pl.kernel(
        _edge_b_body,
        out_type=(
            jax.ShapeDtypeStruct((NPAD, EMB), jnp.float32),
            jax.ShapeDtypeStruct((NPAD, EMB), jnp.float32),
        ),
        mesh=mesh,
        scratch_types=[
            pltpu.VMEM((CHUNK,), jnp.int32),
            pltpu.VMEM((CHUNK, EMB), jnp.float32),
            pltpu.VMEM_SHARED((NPAD, EMB), jnp.float32),
            pltpu.SemaphoreType.DMA,
        ],
    )


def _row(x):
    return x.reshape(1, -1)


def _embed_h(h, pos_enc, sign, at_pad, pos_W, pos_b):
    return pl.pallas_call(
        _embed_h_body,
        grid=(N // NB,),
        in_specs=[
            pl.BlockSpec((NB, 9), lambda i: (i, 0)),
            pl.BlockSpec((NB, PE), lambda i: (i, 0)),
            pl.BlockSpec((1, PE), lambda i: (0, 0)),
            pl.BlockSpec((A_PAD, EMB), lambda i: (0, 0)),
            pl.BlockSpec((PE, EMB), lambda i: (0, 0)),
            pl.BlockSpec((1, EMB), lambda i: (0, 0)),
        ],
        out_specs=pl.BlockSpec((NB, EMB), lambda i: (i, 0)),
        out_shape=jax.ShapeDtypeStruct((N, EMB), jnp.float32),
    )(h, pos_enc, sign, at_pad, pos_W, pos_b)


def _embed_e(e_feat, bt_pad, wc, bc):
    return pl.pallas_call(
        _embed_e_body,
        grid=(E // EB,),
        in_specs=[
            pl.BlockSpec((EB, 3), lambda i: (i, 0)),
            pl.BlockSpec((B_PAD, EMB), lambda i: (0, 0)),
            pl.BlockSpec((EMB, EMB), lambda i: (0, 0)),
            pl.BlockSpec((1, EMB), lambda i: (0, 0)),
        ],
        out_specs=[
            pl.BlockSpec((EB, EMB), lambda i: (i, 0)),
            pl.BlockSpec((EB, EMB), lambda i: (i, 0)),
        ],
        out_shape=[
            jax.ShapeDtypeStruct((E, EMB), jnp.float32),
            jax.ShapeDtypeStruct((E, EMB), jnp.float32),
        ],
    )(e_feat, bt_pad, wc, bc)


def _nodemm(hx, batch2d, vn, wcat, bcat):
    return pl.pallas_call(
        _nodemm_body,
        grid=(N // NB,),
        in_specs=[
            pl.BlockSpec((NB, EMB), lambda i: (i, 0)),
            pl.BlockSpec((NB, 1), lambda i: (i, 0)),
            pl.BlockSpec((G, EMB), lambda i: (0, 0)),
            pl.BlockSpec((EMB, 4 * EMB), lambda i: (0, 0)),
            pl.BlockSpec((1, 4 * EMB), lambda i: (0, 0)),
        ],
        out_specs=[
            pl.BlockSpec((NB, EMB), lambda i: (i, 0)),
            pl.BlockSpec((NB, EMB), lambda i: (i, 0)),
            pl.BlockSpec((NB, 2 * EMB), lambda i: (i, 0)),
            pl.BlockSpec((NB, EMB), lambda i: (i, 0)),
        ],
        out_shape=[
            jax.ShapeDtypeStruct((N, EMB), jnp.float32),
            jax.ShapeDtypeStruct((N, EMB), jnp.float32),
            jax.ShapeDtypeStruct((N, 2 * EMB), jnp.float32),
            jax.ShapeDtypeStruct((N, EMB), jnp.float32),
        ],
    )(hx, batch2d, vn, wcat, bcat)


def _nodeup(heff, a, n0, n1, d0, d1, batch2d, g, b):
    hn, st = pl.pallas_call(
        _nodeup1_body,
        grid=(N // NB,),
        in_specs=[
            pl.BlockSpec((NB, EMB), lambda i: (i, 0)),
            pl.BlockSpec((NB, EMB), lambda i: (i, 0)),
            pl.BlockSpec((NB, EMB), lambda i: (i, 0)),
            pl.BlockSpec((NB, EMB), lambda i: (i, 0)),
            pl.BlockSpec((NB, EMB), lambda i: (i, 0)),
        ],
        out_specs=[
            pl.BlockSpec((NB, EMB), lambda i: (i, 0)),
            pl.BlockSpec((8, EMB), lambda i: (0, 0)),
        ],
        out_shape=[
            jax.ShapeDtypeStruct((N, EMB), jnp.float32),
            jax.ShapeDtypeStruct((8, EMB), jnp.float32),
        ],
    )(a, n0, n1, d0, d1)
    return pl.pallas_call(
        _nodeup2_body,
        grid=(N // NB,),
        in_specs=[
            pl.BlockSpec((NB, EMB), lambda i: (i, 0)),
            pl.BlockSpec((NB, EMB), lambda i: (i, 0)),
            pl.BlockSpec((8, EMB), lambda i: (0, 0)),
            pl.BlockSpec((1, EMB), lambda i: (0, 0)),
            pl.BlockSpec((1, EMB), lambda i: (0, 0)),
            pl.BlockSpec((NB, 1), lambda i: (i, 0)),
        ],
        out_specs=[
            pl.BlockSpec((NB, EMB), lambda i: (i, 0)),
            pl.BlockSpec((G, EMB), lambda i: (0, 0)),
            pl.BlockSpec((G, EMB), lambda i: (0, 0)),
        ],
        out_shape=[
            jax.ShapeDtypeStruct((N, EMB), jnp.float32),
            jax.ShapeDtypeStruct((G, EMB), jnp.float32),
            jax.ShapeDtypeStruct((G, EMB), jnp.float32),
        ],
    )(hn, heff, st, g, b, batch2d)


def _vn_mlp(vn, ps, pc, g, b, w1, b1, w2, b2):
    return pl.pallas_call(
        _vn_body,
        grid=(1,),
        in_specs=[
            pl.BlockSpec((G, EMB), lambda i: (0, 0)),
            pl.BlockSpec((G, EMB), lambda i: (0, 0)),
            pl.BlockSpec((G, EMB), lambda i: (0, 0)),
            pl.BlockSpec((1, EMB), lambda i: (0, 0)),
            pl.BlockSpec((1, EMB), lambda i: (0, 0)),
            pl.BlockSpec((EMB, HID), lambda i: (0, 0)),
            pl.BlockSpec((1, HID), lambda i: (0, 0)),
            pl.BlockSpec((HID, EMB), lambda i: (0, 0)),
            pl.BlockSpec((1, EMB), lambda i: (0, 0)),
        ],
        out_specs=pl.BlockSpec((G, EMB), lambda i: (0, 0)),
        out_shape=jax.ShapeDtypeStruct((G, EMB), jnp.float32),
    )(vn, ps, pc, g, b, w1, b1, w2, b2)


def _edgeup(enew, ex, esum, esq, g, b, wc, bc):
    return pl.pallas_call(
        _edgeup_body,
        grid=(E // EB,),
        in_specs=[
            pl.BlockSpec((EB, EMB), lambda i: (i, 0)),
            pl.BlockSpec((EB, EMB), lambda i: (i, 0)),
            pl.BlockSpec((NWORK, EMB), lambda i: (0, 0)),
            pl.BlockSpec((NWORK, EMB), lambda i: (0, 0)),
            pl.BlockSpec((1, EMB), lambda i: (0, 0)),
            pl.BlockSpec((1, EMB), lambda i: (0, 0)),
            pl.BlockSpec((EMB, EMB), lambda i: (0, 0)),
            pl.BlockSpec((1, EMB), lambda i: (0, 0)),
        ],
        out_specs=[
            pl.BlockSpec((EB, EMB), lambda i: (i, 0)),
            pl.BlockSpec((EB, EMB), lambda i: (i, 0)),
        ],
        out_shape=[
            jax.ShapeDtypeStruct((E, EMB), jnp.float32),
            jax.ShapeDtypeStruct((E, EMB), jnp.float32),
        ],
    )(enew, ex, esum, esq, g, b, wc, bc)


def _edgeup_final(enew, ex, esum, esq, g, b, ebatch2d):
    return pl.pallas_call(
        _edgeup_final_body,
        grid=(E // EB,),
        in_specs=[
            pl.BlockSpec((EB, EMB), lambda i: (i, 0)),
            pl.BlockSpec((EB, EMB), lambda i: (i, 0)),
            pl.BlockSpec((NWORK, EMB), lambda i: (0, 0)),
            pl.BlockSpec((NWORK, EMB), lambda i: (0, 0)),
            pl.BlockSpec((1, EMB), lambda i: (0, 0)),
            pl.BlockSpec((1, EMB), lambda i: (0, 0)),
            pl.BlockSpec((EB, 1), lambda i: (i, 0)),
        ],
        out_specs=[
            pl.BlockSpec((G, EMB), lambda i: (0, 0)),
            pl.BlockSpec((G, EMB), lambda i: (0, 0)),
        ],
        out_shape=[
            jax.ShapeDtypeStruct((G, EMB), jnp.float32),
            jax.ShapeDtypeStruct((G, EMB), jnp.float32),
        ],
    )(enew, ex, esum, esq, g, b, ebatch2d)


def _pred(nps, npc, esum, ecnt, w1, b1, w2, b2):
    return pl.pallas_call(
        _pred_body,
        grid=(1,),
        in_specs=[
            pl.BlockSpec((G, EMB), lambda i: (0, 0)),
            pl.BlockSpec((G, EMB), lambda i: (0, 0)),
            pl.BlockSpec((G, EMB), lambda i: (0, 0)),
            pl.BlockSpec((G, EMB), lambda i: (0, 0)),
            pl.BlockSpec((2 * EMB, HID), lambda i: (0, 0)),
            pl.BlockSpec((1, HID), lambda i: (0, 0)),
            pl.BlockSpec((HID, EMB), lambda i: (0, 0)),
            pl.BlockSpec((1, EMB), lambda i: (0, 0)),
        ],
        out_specs=pl.BlockSpec((G, TASKS), lambda i: (0, 0)),
        out_shape=jax.ShapeDtypeStruct((G, TASKS), jnp.float32),
    )(nps, npc, esum, ecnt, w1, b1, w2, b2)


def kernel(h, e_feat, edge_index, pos_enc, batch_index, atom_table,
           bond_table, pos_W, pos_b, layer_W, layer_b, bn_h_g, bn_h_b,
           bn_e_g, bn_e_b, vn_bn_g, vn_bn_b, vn_W1, vn_b1, vn_W2, vn_b2,
           pred_W1, pred_b1, pred_W2, pred_b2):
    sign = jnp.where(
        jax.random.randint(jax.random.key(42), (1, PE), 0, 2) == 0,
        -1.0, 1.0).astype(jnp.float32)
    at_pad = jnp.zeros((A_PAD, EMB), jnp.float32).at[:atom_table.shape[0]] \
        .set(atom_table)
    bt_pad = jnp.zeros((B_PAD, EMB), jnp.float32).at[:bond_table.shape[0]] \
        .set(bond_table)
    src = edge_index[0].astype(jnp.int32)
    dst = edge_index[1].astype(jnp.int32)
    batch1d = batch_index.astype(jnp.int32)
    batch2d = batch1d.reshape(N, 1)

    hx = _embed_h(h.astype(jnp.int32), pos_enc, sign, at_pad, pos_W,
                  _row(pos_b))
    ex, ce = _embed_e(e_feat.astype(jnp.int32), bt_pad, layer_W[0, 2],
                      _row(layer_b[0, 2]))

    edge_a = {False: _make_edge_a(False), True: _make_edge_a(True)}
    edge_b = _make_edge_b()

    vn = jnp.zeros((G, EMB), jnp.float32)
    for l in range(L):
        wcat = jnp.concatenate(
            [layer_W[l, 0], layer_W[l, 1], layer_W[l, 3], layer_W[l, 4]], 1)
        bcat = jnp.concatenate(
            [layer_b[l, 0], layer_b[l, 1], layer_b[l, 3], layer_b[l, 4]], 0) \
            .reshape(1, 4 * EMB)
        heff, a, bd, eh = _nodemm(hx, batch2d, vn, wcat, bcat)
        last = l == L - 1
        enew, n0, n1, es, eq, eb_out = edge_a[last](
            bd, eh, ce, src, dst, batch1d)
        d0, d1 = edge_b(enew, dst)
        hx, nps, npc = _nodeup(heff, a, n0, n1, d0, d1,
                               batch2d, _row(bn_h_g[l]), _row(bn_h_b[l]))
        if not last:
            vn = _vn_mlp(vn, nps, npc, _row(vn_bn_g[l]), _row(vn_bn_b[l]),
                         vn_W1[l], _row(vn_b1[l]), vn_W2[l], _row(vn_b2[l]))
            ex, ce = _edgeup(enew, ex, es, eq, _row(bn_e_g[l]),
                             _row(bn_e_b[l]), layer_W[l + 1, 2],
                             _row(layer_b[l + 1, 2]))
        else:
            eps, epc = _edgeup_final(enew, ex, es, eq, _row(bn_e_g[l]),
                                     _row(bn_e_b[l]), eb_out.reshape(E, 1))
    return _pred(nps, npc, eps, epc, pred_W1, _row(pred_b1), pred_W2,
                 _row(pred_b2))


# speedup vs baseline: 1.1299x; 1.1299x over previous
import functools

import jax
import jax.numpy as jnp
import numpy as np
from jax import lax
from jax.experimental import pallas as pl
from jax.experimental.pallas import tpu as pltpu
from jax.experimental.pallas import tpu_sc as plsc

N = 10000
E = 160000
EMB = 128
HID = 512
L = 5
G = 256
TASKS = 128
PE = 10
ATOM_DIMS = [119, 5, 12, 12, 10, 6, 6, 2, 2]
BOND_DIMS = [5, 6, 2]
ATOM_OFF = np.concatenate([[0], np.cumsum(ATOM_DIMS)[:-1]]).astype(np.int32)
BOND_OFF = np.concatenate([[0], np.cumsum(BOND_DIMS)[:-1]]).astype(np.int32)
A_PAD = 256
B_PAD = 16

NPAD = 10240
CHUNK = 40
NCHUNK = E // CHUNK
NWORK = 32
CPW = (NCHUNK + NWORK - 1) // NWORK
NSLOT = CPW + (CPW % 2)

NB = 2000
EB = 2000


def _embed_h_body(h_ref, pos_ref, sign_ref, at_ref, pw_ref, pb_ref, out_ref):
    hb = h_ref[...]
    acc = jnp.zeros((NB, A_PAD), jnp.float32)
    for i in range(9):
        col = hb[:, i:i + 1] + np.int32(ATOM_OFF[i])
        ids = lax.broadcasted_iota(jnp.int32, (NB, A_PAD), 1)
        acc = acc + (ids == col).astype(jnp.float32)
    hx = jnp.dot(acc, at_ref[...], preferred_element_type=jnp.float32)
    pos = pos_ref[...] * sign_ref[0:1, :]
    hx = hx + jnp.dot(pos, pw_ref[...], preferred_element_type=jnp.float32)
    out_ref[...] = hx + pb_ref[0:1, :]


def _embed_e_body(ef_ref, bt_ref, wc_ref, bc_ref, ex_ref, ce_ref):
    eb = ef_ref[...]
    acc = jnp.zeros((EB, B_PAD), jnp.float32)
    for i in range(3):
        col = eb[:, i:i + 1] + np.int32(BOND_OFF[i])
        ids = lax.broadcasted_iota(jnp.int32, (EB, B_PAD), 1)
        acc = acc + (ids == col).astype(jnp.float32)
    ex = jnp.dot(acc, bt_ref[...], preferred_element_type=jnp.float32)
    ex_ref[...] = ex
    ce_ref[...] = jnp.dot(ex, wc_ref[...], preferred_element_type=jnp.float32) \
        + bc_ref[0:1, :]


def _nodemm_body(hx_ref, b_ref, vn_ref, w_ref, bias_ref,
                 heff_ref, a_ref, bd_ref, e_ref):
    batch = b_ref[...]
    ids = lax.broadcasted_iota(jnp.int32, (NB, G), 1)
    mask = (ids == batch).astype(jnp.float32)
    heff = hx_ref[...] + jnp.dot(mask, vn_ref[...],
                                 preferred_element_type=jnp.float32)
    p = jnp.dot(heff, w_ref[...], preferred_element_type=jnp.float32) \
        + bias_ref[0:1, :]
    heff_ref[...] = heff
    a_ref[...] = p[:, 0:EMB]
    bd_ref[...] = p[:, EMB:3 * EMB]
    e_ref[...] = p[:, 3 * EMB:4 * EMB]


def _nodeup1_body(a_ref, n0_ref, n1_ref, d0_ref, d1_ref, hn_ref, st_ref):
    num = n0_ref[...] + n1_ref[...]
    den = d0_ref[...] + d1_ref[...] + 1e-6
    hnew = a_ref[...] + num / den
    hn_ref[...] = hnew
    s = jnp.sum(hnew, axis=0, keepdims=True)
    sq = jnp.sum(hnew * hnew, axis=0, keepdims=True)
    part = jnp.concatenate(
        [s, sq, jnp.zeros((6, EMB), jnp.float32)], axis=0)

    @pl.when(pl.program_id(0) == 0)
    def _init():
        st_ref[...] = jnp.zeros_like(st_ref)

    st_ref[...] += part


def _nodeup2_body(hn_ref, heff_ref, st_ref, g_ref, bb_ref, b_ref,
                  hx_ref, ps_ref, pc_ref):
    s = st_ref[...]
    mu = s[0:1, :] / N
    var = s[1:2, :] / N - mu * mu
    hnew = (hn_ref[...] - mu) / jnp.sqrt(var + 1e-5) * g_ref[0:1, :] \
        + bb_ref[0:1, :]
    hx = heff_ref[...] + jnp.maximum(hnew, 0.0)
    hx_ref[...] = hx
    batch = b_ref[...]
    bcol = lax.broadcast_in_dim(batch[:, 0], (G, NB), (1,))
    mask = (bcol == lax.broadcasted_iota(jnp.int32, (G, NB), 0)) \
        .astype(jnp.float32)
    part = jnp.dot(mask, hx, preferred_element_type=jnp.float32)
    cnt = jnp.sum(mask, axis=1, keepdims=True)

    @pl.when(pl.program_id(0) == 0)
    def _init():
        ps_ref[...] = jnp.zeros_like(ps_ref)
        pc_ref[...] = jnp.zeros_like(pc_ref)

    ps_ref[...] += part
    pc_ref[...] += jnp.broadcast_to(cnt, (G, EMB))


def _vn_body(vn_ref, ps_ref, pc_ref, g_ref, b_ref, w1_ref, b1_ref, w2_ref,
             b2_ref, out_ref):
    pool = ps_ref[...] / jnp.maximum(pc_ref[...], 1.0)
    vn = vn_ref[...] + pool
    mu = jnp.mean(vn, axis=0, keepdims=True)
    var = jnp.mean(vn * vn, axis=0, keepdims=True) - mu * mu
    t = (vn - mu) / jnp.sqrt(var + 1e-5) * g_ref[0:1, :] + b_ref[0:1, :]
    t = jnp.maximum(jnp.dot(t, w1_ref[...],
                            preferred_element_type=jnp.float32)
                    + b1_ref[0:1, :], 0.0)
    out_ref[...] = jnp.dot(t, w2_ref[...],
                           preferred_element_type=jnp.float32) + b2_ref[0:1, :]


def _ebn(en, esum_ref, esq_ref, g_ref, b_ref):
    mu = jnp.sum(esum_ref[...], axis=0, keepdims=True) / E
    var = jnp.sum(esq_ref[...], axis=0, keepdims=True) / E - mu * mu
    return jnp.maximum((en - mu) / jnp.sqrt(var + 1e-5) * g_ref[0:1, :]
                       + b_ref[0:1, :], 0.0)


def _edgeup_body(en_ref, ex_ref, esum_ref, esq_ref, g_ref, b_ref,
                 wc_ref, bc_ref, exo_ref, ce_ref):
    ex = ex_ref[...] + _ebn(en_ref[...], esum_ref, esq_ref, g_ref, b_ref)
    exo_ref[...] = ex
    ce_ref[...] = jnp.dot(ex, wc_ref[...],
                          preferred_element_type=jnp.float32) + bc_ref[0:1, :]


def _edgeup_final_body(en_ref, ex_ref, esum_ref, esq_ref, g_ref, b_ref,
                       eb_ref, ps_ref, pc_ref):
    ex = ex_ref[...] + _ebn(en_ref[...], esum_ref, esq_ref, g_ref, b_ref)
    ebatch = eb_ref[...]
    bcol = lax.broadcast_in_dim(ebatch[:, 0], (G, EB), (1,))
    mask = (bcol == lax.broadcasted_iota(jnp.int32, (G, EB), 0)) \
        .astype(jnp.float32)
    part = jnp.dot(mask, ex, preferred_element_type=jnp.float32)
    cnt = jnp.sum(mask, axis=1, keepdims=True)

    @pl.when(pl.program_id(0) == 0)
    def _init():
        ps_ref[...] = jnp.zeros_like(ps_ref)
        pc_ref[...] = jnp.zeros_like(pc_ref)

    ps_ref[...] += part
    pc_ref[...] += jnp.broadcast_to(cnt, (G, EMB))


def _pred_body(nps_ref, npc_ref, es_ref, ec_ref, w1_ref, b1_ref, w2_ref,
               b2_ref, out_ref):
    node_pool = nps_ref[...] / jnp.maximum(npc_ref[...], 1.0)
    epool = es_ref[...] / jnp.maximum(ec_ref[...], 1.0)
    hg = jnp.concatenate([node_pool, epool], axis=-1)
    t = jnp.maximum(jnp.dot(hg, w1_ref[...],
                            preferred_element_type=jnp.float32)
                    + b1_ref[0:1, :], 0.0)
    out_ref[...] = jnp.dot(t, w2_ref[...],
                           preferred_element_type=jnp.float32) + b2_ref[0:1, :]


def _sc_zero_buf(buf):
    def zrow(r, _):
        for j in range(EMB // 16):
            buf[r, pl.ds(j * 16, 16)] = jnp.zeros((16,), jnp.float32)
        return 0
    lax.fori_loop(0, CHUNK, zrow, 0)


def _sc_zero_acc(acc, zbuf, sid):
    _sc_zero_buf(zbuf)
    for t in range(640 // CHUNK):
        pltpu.sync_copy(zbuf, acc.at[pl.ds(sid * 640 + t * CHUNK, CHUNK)])


def _sc_copy_out(acc, cid, sid, out0, out1):
    @pl.when(cid == 0)
    def _():
        pltpu.sync_copy(acc.at[pl.ds(sid * 640, 640)],
                        out0.at[pl.ds(sid * 640, 640)])

    @pl.when(cid == 1)
    def _():
        pltpu.sync_copy(acc.at[pl.ds(sid * 640, 640)],
                        out1.at[pl.ds(sid * 640, 640)])


def _edge_a_body(want_ebatch,
                 bd_hbm, eh_hbm, ce_hbm, src_hbm, dst_hbm, batch_hbm,
                 enew_hbm, num0_hbm, num1_hbm, den0_hbm, den1_hbm,
                 esum_hbm, esq_hbm, ebatch_hbm,
                 srcv0, srcv1, dstv0, dstv1, bdv0, bdv1, ev0, ev1,
                 cv0, cv1, btv, stats_v, acc,
                 semi0, semi1, semg0, semg1):
    cid = lax.axis_index("c")
    sid = lax.axis_index("s")
    wid = sid * 2 + cid
    srcv = (srcv0, srcv1)
    dstv = (dstv0, dstv1)
    bdv = (bdv0, bdv1)
    ev = (ev0, ev1)
    cv = (cv0, cv1)
    semi = (semi0, semi1)
    semg = (semg0, semg1)

    _sc_zero_acc(acc, cv0, sid)
    for r in range(2):
        for j in range(EMB // 16):
            stats_v[r, pl.ds(j * 16, 16)] = jnp.zeros((16,), jnp.float32)
    plsc.subcore_barrier()

    def _pred(i):
        return (wid + NWORK * i) < NCHUNK

    def _base(i):
        return (wid + NWORK * i) * CHUNK

    def _fire_idx(i, b, which):
        @pl.when(_pred(i))
        def _():
            if which in (0, 2):
                pltpu.async_copy(src_hbm.at[pl.ds(_base(i), CHUNK)],
                                 srcv[b], semi[b])
            if which in (1, 2):
                pltpu.async_copy(dst_hbm.at[pl.ds(_base(i), CHUNK)],
                                 dstv[b], semi[b])

    def _fire_gathers(i, b):
        @pl.when(_pred(i))
        def _():
            pltpu.make_async_copy(src_hbm.at[pl.ds(0, CHUNK)], srcv[b],
                                  semi[b]).wait()
            pltpu.make_async_copy(dst_hbm.at[pl.ds(0, CHUNK)], dstv[b],
                                  semi[b]).wait()
            pltpu.async_copy(bd_hbm.at[srcv[b]], bdv[b], semg[b])
            pltpu.async_copy(eh_hbm.at[dstv[b]], ev[b], semg[b])
            pltpu.async_copy(ce_hbm.at[pl.ds(_base(i), CHUNK)], cv[b],
                             semg[b])

    def _consume(i, b):
        @pl.when(_pred(i))
        def _():
            base = _base(i)
            pltpu.make_async_copy(bd_hbm.at[srcv[b]], bdv[b], semg[b]).wait()
            pltpu.make_async_copy(eh_hbm.at[dstv[b]], ev[b], semg[b]).wait()
            pltpu.make_async_copy(ce_hbm.at[pl.ds(base, CHUNK)], cv[b],
                                  semg[b]).wait()

            zeros16 = jnp.zeros((16,), jnp.float32)
            carry0 = (tuple(zeros16 for _ in range(EMB // 16)),
                      tuple(zeros16 for _ in range(EMB // 16)))

            @plsc.parallel_loop(0, CHUNK, unroll=4, carry=carry0)
            def _rows(r, cstat):
                s0, s1 = cstat
                n0, n1 = [], []
                for j in range(EMB // 16):
                    sl = pl.ds(j * 16, 16)
                    e = cv[b][r, sl] + bdv[b][r, pl.ds(EMB + j * 16, 16)] \
                        + ev[b][r, sl]
                    cv[b][r, sl] = e
                    sg = 1.0 / (1.0 + jnp.exp(-e))
                    ev[b][r, sl] = sg * bdv[b][r, sl]
                    n0.append(s0[j] + e)
                    n1.append(s1[j] + e * e)
                return (tuple(n0), tuple(n1))

            s0, s1 = _rows
            for j in range(EMB // 16):
                sl = pl.ds(j * 16, 16)
                stats_v[0, sl] = stats_v[0, sl] + s0[j]
                stats_v[1, sl] = stats_v[1, sl] + s1[j]
            pltpu.sync_copy(cv[b], enew_hbm.at[pl.ds(base, CHUNK)])
            pltpu.sync_copy(ev[b], acc.at[dstv[b]], add=True)
            if want_ebatch:
                pltpu.async_copy(batch_hbm.at[dstv[b]], btv, semg[b]).wait()
                pltpu.sync_copy(btv, ebatch_hbm.at[pl.ds(base, CHUNK)])

    _fire_idx(0, 0, 2)
    _fire_idx(1, 1, 2)
    _fire_gathers(0, 0)

    def slot_pair(io, _):
        for b in range(2):
            i = 2 * io + b
            _fire_gathers(i + 1, 1 - b)
            _consume(i, b)
            _fire_idx(i + 2, b, 0)
            _fire_idx(i + 2, b, 1)
        return 0

    lax.fori_loop(0, NSLOT // 2, slot_pair, 0)
    pltpu.sync_copy(stats_v.at[0], esum_hbm.at[wid])
    pltpu.sync_copy(stats_v.at[1], esq_hbm.at[wid])
    plsc.subcore_barrier()
    _sc_copy_out(acc, cid, sid, num0_hbm, num1_hbm)

    _sc_zero_acc(acc, cv0, sid)
    plsc.subcore_barrier()

    def _fire2(i, b):
        @pl.when(_pred(i))
        def _():
            base = _base(i)
            pltpu.async_copy(enew_hbm.at[pl.ds(base, CHUNK)], cv[b], semg[b])
            pltpu.async_copy(dst_hbm.at[pl.ds(base, CHUNK)], dstv[b], semg[b])

    def _consume2(i, b):
        @pl.when(_pred(i))
        def _():
            pltpu.make_async_copy(enew_hbm.at[pl.ds(0, CHUNK)], cv[b],
                                  semg[b]).wait()
            pltpu.make_async_copy(dst_hbm.at[pl.ds(0, CHUNK)], dstv[b],
                                  semg[b]).wait()

            @plsc.parallel_loop(0, CHUNK, unroll=4)
            def _rows2(r):
                for j in range(EMB // 16):
                    sl = pl.ds(j * 16, 16)
                    cv[b][r, sl] = 1.0 / (1.0 + jnp.exp(-cv[b][r, sl]))

            pltpu.sync_copy(cv[b], acc.at[dstv[b]], add=True)

    _fire2(0, 0)

    def slot_pair2(io, _):
        for b in range(2):
            i = 2 * io + b
            _fire2(i + 1, 1 - b)
            _consume2(i, b)
        return 0

    lax.fori_loop(0, NSLOT // 2, slot_pair2, 0)
    plsc.subcore_barrier()
    _sc_copy_out(acc, cid, sid, den0_hbm, den1_hbm)


def _make_edge_a(want_ebatch):
    mesh = plsc.VectorSubcoreMesh(core_axis_name="c", subcore_axis_name="s")
    return pl.kernel(
        functools.partial(_edge_a_body, want_ebatch),
        out_type=(
            jax.ShapeDtypeStruct((E, EMB), jnp.float32),
            jax.ShapeDtypeStruct((NPAD, EMB), jnp.float32),
            jax.ShapeDtypeStruct((NPAD, EMB), jnp.float32),
            jax.ShapeDtypeStruct((NPAD, EMB), jnp.float32),
            jax.ShapeDtypeStruct((NPAD, EMB), jnp.float32),
            jax.ShapeDtypeStruct((NWORK, EMB), jnp.float32),
            jax.ShapeDtypeStruct((NWORK, EMB), jnp.float32),
            jax.ShapeDtypeStruct((E,), jnp.int32),
        ),
        mesh=mesh,
        scratch_types=[
            pltpu.VMEM((CHUNK,), jnp.int32),
            pltpu.VMEM((CHUNK,), jnp.int32),
            pltpu.VMEM((CHUNK,), jnp.int32),
            pltpu.VMEM((CHUNK,), jnp.int32),
            pltpu.VMEM((CHUNK, 2 * EMB), jnp.float32),
            pltpu.VMEM((CHUNK, 2 * EMB), jnp.float32),
            pltpu.VMEM((CHUNK, EMB), jnp.float32),
            pltpu.VMEM((CHUNK, EMB), jnp.float32),
            pltpu.VMEM((CHUNK, EMB), jnp.float32),
            pltpu.VMEM((CHUNK, EMB), jnp.float32),
            pltpu.VMEM((CHUNK,), jnp.int32),
            pltpu.VMEM((2, EMB), jnp.float32),
            pltpu.VMEM_SHARED((NPAD, EMB), jnp.float32),
            pltpu.SemaphoreType.DMA,
            pltpu.SemaphoreType.DMA,
            pltpu.SemaphoreType.DMA,
            pltpu.SemaphoreType.DMA,
        ],
    )


def _row(x):
    return x.reshape(1, -1)


def _embed_h(h, pos_enc, sign, at_pad, pos_W, pos_b):
    return pl.pallas_call(
        _embed_h_body,
        grid=(N // NB,),
        in_specs=[
            pl.BlockSpec((NB, 9), lambda i: (i, 0)),
            pl.BlockSpec((NB, PE), lambda i: (i, 0)),
            pl.BlockSpec((1, PE), lambda i: (0, 0)),
            pl.BlockSpec((A_PAD, EMB), lambda i: (0, 0)),
            pl.BlockSpec((PE, EMB), lambda i: (0, 0)),
            pl.BlockSpec((1, EMB), lambda i: (0, 0)),
        ],
        out_specs=pl.BlockSpec((NB, EMB), lambda i: (i, 0)),
        out_shape=jax.ShapeDtypeStruct((N, EMB), jnp.float32),
    )(h, pos_enc, sign, at_pad, pos_W, pos_b)


def _embed_e(e_feat, bt_pad, wc, bc):
    return pl.pallas_call(
        _embed_e_body,
        grid=(E // EB,),
        in_specs=[
            pl.BlockSpec((EB, 3), lambda i: (i, 0)),
            pl.BlockSpec((B_PAD, EMB), lambda i: (0, 0)),
            pl.BlockSpec((EMB, EMB), lambda i: (0, 0)),
            pl.BlockSpec((1, EMB), lambda i: (0, 0)),
        ],
        out_specs=[
            pl.BlockSpec((EB, EMB), lambda i: (i, 0)),
            pl.BlockSpec((EB, EMB), lambda i: (i, 0)),
        ],
        out_shape=[
            jax.ShapeDtypeStruct((E, EMB), jnp.float32),
            jax.ShapeDtypeStruct((E, EMB), jnp.float32),
        ],
    )(e_feat, bt_pad, wc, bc)


def _nodemm(hx, batch2d, vn, wcat, bcat):
    return pl.pallas_call(
        _nodemm_body,
        grid=(N // NB,),
        in_specs=[
            pl.BlockSpec((NB, EMB), lambda i: (i, 0)),
            pl.BlockSpec((NB, 1), lambda i: (i, 0)),
            pl.BlockSpec((G, EMB), lambda i: (0, 0)),
            pl.BlockSpec((EMB, 4 * EMB), lambda i: (0, 0)),
            pl.BlockSpec((1, 4 * EMB), lambda i: (0, 0)),
        ],
        out_specs=[
            pl.BlockSpec((NB, EMB), lambda i: (i, 0)),
            pl.BlockSpec((NB, EMB), lambda i: (i, 0)),
            pl.BlockSpec((NB, 2 * EMB), lambda i: (i, 0)),
            pl.BlockSpec((NB, EMB), lambda i: (i, 0)),
        ],
        out_shape=[
            jax.ShapeDtypeStruct((N, EMB), jnp.float32),
            jax.ShapeDtypeStruct((N, EMB), jnp.float32),
            jax.ShapeDtypeStruct((N, 2 * EMB), jnp.float32),
            jax.ShapeDtypeStruct((N, EMB), jnp.float32),
        ],
    )(hx, batch2d, vn, wcat, bcat)


def _nodeup(heff, a, n0, n1, d0, d1, batch2d, g, b):
    hn, st = pl.pallas_call(
        _nodeup1_body,
        grid=(N // NB,),
        in_specs=[
            pl.BlockSpec((NB, EMB), lambda i: (i, 0)),
            pl.BlockSpec((NB, EMB), lambda i: (i, 0)),
            pl.BlockSpec((NB, EMB), lambda i: (i, 0)),
            pl.BlockSpec((NB, EMB), lambda i: (i, 0)),
            pl.BlockSpec((NB, EMB), lambda i: (i, 0)),
        ],
        out_specs=[
            pl.BlockSpec((NB, EMB), lambda i: (i, 0)),
            pl.BlockSpec((8, EMB), lambda i: (0, 0)),
        ],
        out_shape=[
            jax.ShapeDtypeStruct((N, EMB), jnp.float32),
            jax.ShapeDtypeStruct((8, EMB), jnp.float32),
        ],
    )(a, n0, n1, d0, d1)
    return pl.pallas_call(
        _nodeup2_body,
        grid=(N // NB,),
        in_specs=[
            pl.BlockSpec((NB, EMB), lambda i: (i, 0)),
            pl.BlockSpec((NB, EMB), lambda i: (i, 0)),
            pl.BlockSpec((8, EMB), lambda i: (0, 0)),
            pl.BlockSpec((1, EMB), lambda i: (0, 0)),
            pl.BlockSpec((1, EMB), lambda i: (0, 0)),
            pl.BlockSpec((NB, 1), lambda i: (i, 0)),
        ],
        out_specs=[
            pl.BlockSpec((NB, EMB), lambda i: (i, 0)),
            pl.BlockSpec((G, EMB), lambda i: (0, 0)),
            pl.BlockSpec((G, EMB), lambda i: (0, 0)),
        ],
        out_shape=[
            jax.ShapeDtypeStruct((N, EMB), jnp.float32),
            jax.ShapeDtypeStruct((G, EMB), jnp.float32),
            jax.ShapeDtypeStruct((G, EMB), jnp.float32),
        ],
    )(hn, heff, st, g, b, batch2d)


def _vn_mlp(vn, ps, pc, g, b, w1, b1, w2, b2):
    return pl.pallas_call(
        _vn_body,
        grid=(1,),
        in_specs=[
            pl.BlockSpec((G, EMB), lambda i: (0, 0)),
            pl.BlockSpec((G, EMB), lambda i: (0, 0)),
            pl.BlockSpec((G, EMB), lambda i: (0, 0)),
            pl.BlockSpec((1, EMB), lambda i: (0, 0)),
            pl.BlockSpec((1, EMB), lambda i: (0, 0)),
            pl.BlockSpec((EMB, HID), lambda i: (0, 0)),
            pl.BlockSpec((1, HID), lambda i: (0, 0)),
            pl.BlockSpec((HID, EMB), lambda i: (0, 0)),
            pl.BlockSpec((1, EMB), lambda i: (0, 0)),
        ],
        out_specs=pl.BlockSpec((G, EMB), lambda i: (0, 0)),
        out_shape=jax.ShapeDtypeStruct((G, EMB), jnp.float32),
    )(vn, ps, pc, g, b, w1, b1, w2, b2)


def _edgeup(enew, ex, esum, esq, g, b, wc, bc):
    return pl.pallas_call(
        _edgeup_body,
        grid=(E // EB,),
        in_specs=[
            pl.BlockSpec((EB, EMB), lambda i: (i, 0)),
            pl.BlockSpec((EB, EMB), lambda i: (i, 0)),
            pl.BlockSpec((NWORK, EMB), lambda i: (0, 0)),
            pl.BlockSpec((NWORK, EMB), lambda i: (0, 0)),
            pl.BlockSpec((1, EMB), lambda i: (0, 0)),
            pl.BlockSpec((1, EMB), lambda i: (0, 0)),
            pl.BlockSpec((EMB, EMB), lambda i: (0, 0)),
            pl.BlockSpec((1, EMB), lambda i: (0, 0)),
        ],
        out_specs=[
            pl.BlockSpec((EB, EMB), lambda i: (i, 0)),
            pl.BlockSpec((EB, EMB), lambda i: (i, 0)),
        ],
        out_shape=[
            jax.ShapeDtypeStruct((E, EMB), jnp.float32),
            jax.ShapeDtypeStruct((E, EMB), jnp.float32),
        ],
    )(enew, ex, esum, esq, g, b, wc, bc)


def _edgeup_final(enew, ex, esum, esq, g, b, ebatch2d):
    return pl.pallas_call(
        _edgeup_final_body,
        grid=(E // EB,),
        in_specs=[
            pl.BlockSpec((EB, EMB), lambda i: (i, 0)),
            pl.BlockSpec((EB, EMB), lambda i: (i, 0)),
            pl.BlockSpec((NWORK, EMB), lambda i: (0, 0)),
            pl.BlockSpec((NWORK, EMB), lambda i: (0, 0)),
            pl.BlockSpec((1, EMB), lambda i: (0, 0)),
            pl.BlockSpec((1, EMB), lambda i: (0, 0)),
            pl.BlockSpec((EB, 1), lambda i: (i, 0)),
        ],
        out_specs=[
            pl.BlockSpec((G, EMB), lambda i: (0, 0)),
            pl.BlockSpec((G, EMB), lambda i: (0, 0)),
        ],
        out_shape=[
            jax.ShapeDtypeStruct((G, EMB), jnp.float32),
            jax.ShapeDtypeStruct((G, EMB), jnp.float32),
        ],
    )(enew, ex, esum, esq, g, b, ebatch2d)


def _pred(nps, npc, esum, ecnt, w1, b1, w2, b2):
    return pl.pallas_call(
        _pred_body,
        grid=(1,),
        in_specs=[
            pl.BlockSpec((G, EMB), lambda i: (0, 0)),
            pl.BlockSpec((G, EMB), lambda i: (0, 0)),
            pl.BlockSpec((G, EMB), lambda i: (0, 0)),
            pl.BlockSpec((G, EMB), lambda i: (0, 0)),
            pl.BlockSpec((2 * EMB, HID), lambda i: (0, 0)),
            pl.BlockSpec((1, HID), lambda i: (0, 0)),
            pl.BlockSpec((HID, EMB), lambda i: (0, 0)),
            pl.BlockSpec((1, EMB), lambda i: (0, 0)),
        ],
        out_specs=pl.BlockSpec((G, TASKS), lambda i: (0, 0)),
        out_shape=jax.ShapeDtypeStruct((G, TASKS), jnp.float32),
    )(nps, npc, esum, ecnt, w1, b1, w2, b2)


def kernel(h, e_feat, edge_index, pos_enc, batch_index, atom_table,
           bond_table, pos_W, pos_b, layer_W, layer_b, bn_h_g, bn_h_b,
           bn_e_g, bn_e_b, vn_bn_g, vn_bn_b, vn_W1, vn_b1, vn_W2, vn_b2,
           pred_W1, pred_b1, pred_W2, pred_b2):
    sign = jnp.where(
        jax.random.randint(jax.random.key(42), (1, PE), 0, 2) == 0,
        -1.0, 1.0).astype(jnp.float32)
    at_pad = jnp.zeros((A_PAD, EMB), jnp.float32).at[:atom_table.shape[0]] \
        .set(atom_table)
    bt_pad = jnp.zeros((B_PAD, EMB), jnp.float32).at[:bond_table.shape[0]] \
        .set(bond_table)
    src = edge_index[0].astype(jnp.int32)
    dst = edge_index[1].astype(jnp.int32)
    batch1d = batch_index.astype(jnp.int32)
    batch2d = batch1d.reshape(N, 1)

    hx = _embed_h(h.astype(jnp.int32), pos_enc, sign, at_pad, pos_W,
                  _row(pos_b))
    ex, ce = _embed_e(e_feat.astype(jnp.int32), bt_pad, layer_W[0, 2],
                      _row(layer_b[0, 2]))

    edge_a = {False: _make_edge_a(False), True: _make_edge_a(True)}

    vn = jnp.zeros((G, EMB), jnp.float32)
    for l in range(L):
        wcat = jnp.concatenate(
            [layer_W[l, 0], layer_W[l, 1], layer_W[l, 3], layer_W[l, 4]], 1)
        bcat = jnp.concatenate(
            [layer_b[l, 0], layer_b[l, 1], layer_b[l, 3], layer_b[l, 4]], 0) \
            .reshape(1, 4 * EMB)
        heff, a, bd, eh = _nodemm(hx, batch2d, vn, wcat, bcat)
        last = l == L - 1
        enew, n0, n1, d0, d1, es, eq, eb_out = edge_a[last](
            bd, eh, ce, src, dst, batch1d)
        hx, nps, npc = _nodeup(heff, a, n0, n1, d0, d1,
                               batch2d, _row(bn_h_g[l]), _row(bn_h_b[l]))
        if not last:
            vn = _vn_mlp(vn, nps, npc, _row(vn_bn_g[l]), _row(vn_bn_b[l]),
                         vn_W1[l], _row(vn_b1[l]), vn_W2[l], _row(vn_b2[l]))
            ex, ce = _edgeup(enew, ex, es, eq, _row(bn_e_g[l]),
                             _row(bn_e_b[l]), layer_W[l + 1, 2],
                             _row(layer_b[l + 1, 2]))
        else:
            eps, epc = _edgeup_final(enew, ex, es, eq, _row(bn_e_g[l]),
                                     _row(bn_e_b[l]), eb_out.reshape(E, 1))
    return _pred(nps, npc, eps, epc, pred_W1, _row(pred_b1), pred_W2,
                 _row(pred_b2))


# speedup vs baseline: 2.2460x; 1.9878x over previous
import functools

import jax
import jax.numpy as jnp
import numpy as np
from jax import lax
from jax.experimental import pallas as pl
from jax.experimental.pallas import tpu as pltpu
from jax.experimental.pallas import tpu_sc as plsc

N = 10000
E = 160000
EMB = 128
HID = 512
L = 5
G = 256
TASKS = 128
PE = 10
ATOM_DIMS = [119, 5, 12, 12, 10, 6, 6, 2, 2]
BOND_DIMS = [5, 6, 2]
ATOM_OFF = np.concatenate([[0], np.cumsum(ATOM_DIMS)[:-1]]).astype(np.int32)
BOND_OFF = np.concatenate([[0], np.cumsum(BOND_DIMS)[:-1]]).astype(np.int32)
A_PAD = 256
B_PAD = 16

NPAD = 10240
CHUNK = 40
NCHUNK = E // CHUNK
NWORK = 32
CPW = (NCHUNK + NWORK - 1) // NWORK
NSLOT = CPW + (CPW % 2)

NB = 2000
EB = 3200


def _embed_h_body(h_ref, pos_ref, sign_ref, at_ref, pw_ref, pb_ref, out_ref):
    hb = h_ref[...]
    acc = jnp.zeros((NB, A_PAD), jnp.float32)
    for i in range(9):
        col = hb[:, i:i + 1] + np.int32(ATOM_OFF[i])
        ids = lax.broadcasted_iota(jnp.int32, (NB, A_PAD), 1)
        acc = acc + (ids == col).astype(jnp.float32)
    hx = jnp.dot(acc, at_ref[...], preferred_element_type=jnp.float32)
    pos = pos_ref[...] * sign_ref[0:1, :]
    hx = hx + jnp.dot(pos, pw_ref[...], preferred_element_type=jnp.float32)
    out_ref[...] = hx + pb_ref[0:1, :]


def _embed_e_body(ef_ref, bt_ref, wc_ref, bc_ref, ex_ref, ce_ref):
    eb = ef_ref[...]
    acc = jnp.zeros((EB, B_PAD), jnp.float32)
    for i in range(3):
        col = eb[:, i:i + 1] + np.int32(BOND_OFF[i])
        ids = lax.broadcasted_iota(jnp.int32, (EB, B_PAD), 1)
        acc = acc + (ids == col).astype(jnp.float32)
    ex = jnp.dot(acc, bt_ref[...], preferred_element_type=jnp.float32)
    ex_ref[...] = ex
    ce_ref[...] = jnp.dot(ex, wc_ref[...], preferred_element_type=jnp.float32) \
        + bc_ref[0:1, :]


def _nodemm_body(hx_ref, b_ref, vn_ref, w_ref, bias_ref,
                 heff_ref, a_ref, bd_ref, e_ref):
    batch = b_ref[...]
    ids = lax.broadcasted_iota(jnp.int32, (NB, G), 1)
    mask = (ids == batch).astype(jnp.float32)
    heff = hx_ref[...] + jnp.dot(mask, vn_ref[...],
                                 preferred_element_type=jnp.float32)
    p = jnp.dot(heff, w_ref[...], preferred_element_type=jnp.float32) \
        + bias_ref[0:1, :]
    heff_ref[...] = heff
    a_ref[...] = p[:, 0:EMB]
    bd_ref[...] = p[:, EMB:3 * EMB]
    e_ref[...] = p[:, 3 * EMB:4 * EMB]


def _nodeup1_body(a_ref, n0_ref, n1_ref, d0_ref, d1_ref, hn_ref, st_ref):
    num = n0_ref[...] + n1_ref[...]
    den = d0_ref[...] + d1_ref[...] + 1e-6
    hnew = a_ref[...] + num / den
    hn_ref[...] = hnew
    s = jnp.sum(hnew, axis=0, keepdims=True)
    sq = jnp.sum(hnew * hnew, axis=0, keepdims=True)
    part = jnp.concatenate(
        [s, sq, jnp.zeros((6, EMB), jnp.float32)], axis=0)

    @pl.when(pl.program_id(0) == 0)
    def _init():
        st_ref[...] = jnp.zeros_like(st_ref)

    st_ref[...] += part


def _nodeup2_body(hn_ref, heff_ref, st_ref, g_ref, bb_ref, b_ref,
                  hx_ref, ps_ref, pc_ref):
    s = st_ref[...]
    mu = s[0:1, :] / N
    var = s[1:2, :] / N - mu * mu
    hnew = (hn_ref[...] - mu) / jnp.sqrt(var + 1e-5) * g_ref[0:1, :] \
        + bb_ref[0:1, :]
    hx = heff_ref[...] + jnp.maximum(hnew, 0.0)
    hx_ref[...] = hx
    ids = lax.broadcasted_iota(jnp.int32, (NB, G), 1)
    maskn = (ids == b_ref[...]).astype(jnp.float32)
    dn = (((0,), (0,)), ((), ()))
    part = lax.dot_general(maskn, hx, dn,
                           preferred_element_type=jnp.float32)
    cntp = lax.dot_general(maskn, jnp.ones((NB, EMB), jnp.float32), dn,
                           preferred_element_type=jnp.float32)

    @pl.when(pl.program_id(0) == 0)
    def _init():
        ps_ref[...] = jnp.zeros_like(ps_ref)
        pc_ref[...] = jnp.zeros_like(pc_ref)

    ps_ref[...] += part
    pc_ref[...] += cntp


def _vn_body(vn_ref, ps_ref, pc_ref, g_ref, b_ref, w1_ref, b1_ref, w2_ref,
             b2_ref, out_ref):
    pool = ps_ref[...] / jnp.maximum(pc_ref[...], 1.0)
    vn = vn_ref[...] + pool
    mu = jnp.mean(vn, axis=0, keepdims=True)
    var = jnp.mean(vn * vn, axis=0, keepdims=True) - mu * mu
    t = (vn - mu) / jnp.sqrt(var + 1e-5) * g_ref[0:1, :] + b_ref[0:1, :]
    t = jnp.maximum(jnp.dot(t, w1_ref[...],
                            preferred_element_type=jnp.float32)
                    + b1_ref[0:1, :], 0.0)
    out_ref[...] = jnp.dot(t, w2_ref[...],
                           preferred_element_type=jnp.float32) + b2_ref[0:1, :]


def _ebn(en, esum_ref, esq_ref, g_ref, b_ref):
    mu = jnp.sum(esum_ref[...], axis=0, keepdims=True) / E
    var = jnp.sum(esq_ref[...], axis=0, keepdims=True) / E - mu * mu
    return jnp.maximum((en - mu) / jnp.sqrt(var + 1e-5) * g_ref[0:1, :]
                       + b_ref[0:1, :], 0.0)


def _edgeup_body(en_ref, ex_ref, esum_ref, esq_ref, g_ref, b_ref,
                 wc_ref, bc_ref, exo_ref, ce_ref):
    ex = ex_ref[...] + _ebn(en_ref[...], esum_ref, esq_ref, g_ref, b_ref)
    exo_ref[...] = ex
    ce_ref[...] = jnp.dot(ex, wc_ref[...],
                          preferred_element_type=jnp.float32) + bc_ref[0:1, :]


def _edgeup_final_body(en_ref, ex_ref, esum_ref, esq_ref, g_ref, b_ref,
                       eb_ref, ps_ref, pc_ref):
    ex = ex_ref[...] + _ebn(en_ref[...], esum_ref, esq_ref, g_ref, b_ref)
    bcol = jnp.broadcast_to(eb_ref[0:1, :], (G, EB))
    mask = (bcol == lax.broadcasted_iota(jnp.int32, (G, EB), 0)) \
        .astype(jnp.float32)
    part = jnp.dot(mask, ex, preferred_element_type=jnp.float32)
    cnt = jnp.sum(mask, axis=1, keepdims=True)

    @pl.when(pl.program_id(0) == 0)
    def _init():
        ps_ref[...] = jnp.zeros_like(ps_ref)
        pc_ref[...] = jnp.zeros_like(pc_ref)

    ps_ref[...] += part
    pc_ref[...] += jnp.broadcast_to(cnt, (G, EMB))


def _pred_body(nps_ref, npc_ref, es_ref, ec_ref, w1_ref, b1_ref, w2_ref,
               b2_ref, out_ref):
    node_pool = nps_ref[...] / jnp.maximum(npc_ref[...], 1.0)
    epool = es_ref[...] / jnp.maximum(ec_ref[...], 1.0)
    hg = jnp.concatenate([node_pool, epool], axis=-1)
    t = jnp.maximum(jnp.dot(hg, w1_ref[...],
                            preferred_element_type=jnp.float32)
                    + b1_ref[0:1, :], 0.0)
    out_ref[...] = jnp.dot(t, w2_ref[...],
                           preferred_element_type=jnp.float32) + b2_ref[0:1, :]


def _sc_zero_buf(buf):
    def zrow(r, _):
        for j in range(EMB // 16):
            buf[r, pl.ds(j * 16, 16)] = jnp.zeros((16,), jnp.float32)
        return 0
    lax.fori_loop(0, CHUNK, zrow, 0)


def _sc_zero_acc(acc, zbuf, sid):
    _sc_zero_buf(zbuf)
    for t in range(640 // CHUNK):
        pltpu.sync_copy(zbuf, acc.at[pl.ds(sid * 640 + t * CHUNK, CHUNK)])


def _sc_copy_out(acc, cid, sid, out0, out1):
    @pl.when(cid == 0)
    def _():
        pltpu.sync_copy(acc.at[pl.ds(sid * 640, 640)],
                        out0.at[pl.ds(sid * 640, 640)])

    @pl.when(cid == 1)
    def _():
        pltpu.sync_copy(acc.at[pl.ds(sid * 640, 640)],
                        out1.at[pl.ds(sid * 640, 640)])


def _edge_a_body(want_ebatch,
                 bd_hbm, eh_hbm, ce_hbm, src_hbm, dst_hbm, batch_hbm,
                 enew_hbm, num0_hbm, num1_hbm, den0_hbm, den1_hbm,
                 esum_hbm, esq_hbm, ebatch_hbm,
                 srcv0, srcv1, dstv0, dstv1, bdv0, bdv1, ev0, ev1,
                 cv0, cv1, btv, stats_v, acc,
                 semi0, semi1, semg0, semg1):
    cid = lax.axis_index("c")
    sid = lax.axis_index("s")
    wid = sid * 2 + cid
    srcv = (srcv0, srcv1)
    dstv = (dstv0, dstv1)
    bdv = (bdv0, bdv1)
    ev = (ev0, ev1)
    cv = (cv0, cv1)
    semi = (semi0, semi1)
    semg = (semg0, semg1)

    _sc_zero_acc(acc, cv0, sid)
    for r in range(2):
        for j in range(EMB // 16):
            stats_v[r, pl.ds(j * 16, 16)] = jnp.zeros((16,), jnp.float32)
    plsc.subcore_barrier()

    def _pred(i):
        return (wid + NWORK * i) < NCHUNK

    def _base(i):
        return (wid + NWORK * i) * CHUNK

    def _fire_idx(i, b, which):
        @pl.when(_pred(i))
        def _():
            if which in (0, 2):
                pltpu.async_copy(src_hbm.at[pl.ds(_base(i), CHUNK)],
                                 srcv[b], semi[b])
            if which in (1, 2):
                pltpu.async_copy(dst_hbm.at[pl.ds(_base(i), CHUNK)],
                                 dstv[b], semi[b])

    def _fire_gathers(i, b):
        @pl.when(_pred(i))
        def _():
            pltpu.make_async_copy(src_hbm.at[pl.ds(0, CHUNK)], srcv[b],
                                  semi[b]).wait()
            pltpu.make_async_copy(dst_hbm.at[pl.ds(0, CHUNK)], dstv[b],
                                  semi[b]).wait()
            pltpu.async_copy(bd_hbm.at[srcv[b]], bdv[b], semg[b])
            pltpu.async_copy(eh_hbm.at[dstv[b]], ev[b], semg[b])
            pltpu.async_copy(ce_hbm.at[pl.ds(_base(i), CHUNK)], cv[b],
                             semg[b])

    def _consume(i, b):
        @pl.when(_pred(i))
        def _():
            base = _base(i)
            pltpu.make_async_copy(bd_hbm.at[srcv[b]], bdv[b], semg[b]).wait()
            pltpu.make_async_copy(eh_hbm.at[dstv[b]], ev[b], semg[b]).wait()
            pltpu.make_async_copy(ce_hbm.at[pl.ds(base, CHUNK)], cv[b],
                                  semg[b]).wait()

            zeros16 = jnp.zeros((16,), jnp.float32)
            carry0 = (tuple(zeros16 for _ in range(EMB // 16)),
                      tuple(zeros16 for _ in range(EMB // 16)))

            @plsc.parallel_loop(0, CHUNK, unroll=4, carry=carry0)
            def _rows(r, cstat):
                s0, s1 = cstat
                n0, n1 = [], []
                for j in range(EMB // 16):
                    sl = pl.ds(j * 16, 16)
                    e = cv[b][r, sl] + bdv[b][r, pl.ds(EMB + j * 16, 16)] \
                        + ev[b][r, sl]
                    cv[b][r, sl] = e
                    sg = 1.0 / (1.0 + jnp.exp(-e))
                    ev[b][r, sl] = sg * bdv[b][r, sl]
                    n0.append(s0[j] + e)
                    n1.append(s1[j] + e * e)
                return (tuple(n0), tuple(n1))

            s0, s1 = _rows
            for j in range(EMB // 16):
                sl = pl.ds(j * 16, 16)
                stats_v[0, sl] = stats_v[0, sl] + s0[j]
                stats_v[1, sl] = stats_v[1, sl] + s1[j]
            pltpu.sync_copy(cv[b], enew_hbm.at[pl.ds(base, CHUNK)])
            pltpu.sync_copy(ev[b], acc.at[dstv[b]], add=True)
            if want_ebatch:
                pltpu.async_copy(batch_hbm.at[dstv[b]], btv, semg[b]).wait()
                pltpu.sync_copy(btv, ebatch_hbm.at[pl.ds(base, CHUNK)])

    _fire_idx(0, 0, 2)
    _fire_idx(1, 1, 2)
    _fire_gathers(0, 0)

    def slot_pair(io, _):
        for b in range(2):
            i = 2 * io + b
            _fire_gathers(i + 1, 1 - b)
            _consume(i, b)
            _fire_idx(i + 2, b, 0)
            _fire_idx(i + 2, b, 1)
        return 0

    lax.fori_loop(0, NSLOT // 2, slot_pair, 0)
    pltpu.sync_copy(stats_v.at[0], esum_hbm.at[wid])
    pltpu.sync_copy(stats_v.at[1], esq_hbm.at[wid])
    plsc.subcore_barrier()
    _sc_copy_out(acc, cid, sid, num0_hbm, num1_hbm)

    _sc_zero_acc(acc, cv0, sid)
    plsc.subcore_barrier()

    def _fire2(i, b):
        @pl.when(_pred(i))
        def _():
            base = _base(i)
            pltpu.async_copy(enew_hbm.at[pl.ds(base, CHUNK)], cv[b], semg[b])
            pltpu.async_copy(dst_hbm.at[pl.ds(base, CHUNK)], dstv[b], semg[b])

    def _consume2(i, b):
        @pl.when(_pred(i))
        def _():
            pltpu.make_async_copy(enew_hbm.at[pl.ds(0, CHUNK)], cv[b],
                                  semg[b]).wait()
            pltpu.make_async_copy(dst_hbm.at[pl.ds(0, CHUNK)], dstv[b],
                                  semg[b]).wait()

            @plsc.parallel_loop(0, CHUNK, unroll=4)
            def _rows2(r):
                for j in range(EMB // 16):
                    sl = pl.ds(j * 16, 16)
                    cv[b][r, sl] = 1.0 / (1.0 + jnp.exp(-cv[b][r, sl]))

            pltpu.sync_copy(cv[b], acc.at[dstv[b]], add=True)

    _fire2(0, 0)

    def slot_pair2(io, _):
        for b in range(2):
            i = 2 * io + b
            _fire2(i + 1, 1 - b)
            _consume2(i, b)
        return 0

    lax.fori_loop(0, NSLOT // 2, slot_pair2, 0)
    plsc.subcore_barrier()
    _sc_copy_out(acc, cid, sid, den0_hbm, den1_hbm)


def _make_edge_a(want_ebatch):
    mesh = plsc.VectorSubcoreMesh(core_axis_name="c", subcore_axis_name="s")
    return pl.kernel(
        functools.partial(_edge_a_body, want_ebatch),
        out_type=(
            jax.ShapeDtypeStruct((E, EMB), jnp.float32),
            jax.ShapeDtypeStruct((NPAD, EMB), jnp.float32),
            jax.ShapeDtypeStruct((NPAD, EMB), jnp.float32),
            jax.ShapeDtypeStruct((NPAD, EMB), jnp.float32),
            jax.ShapeDtypeStruct((NPAD, EMB), jnp.float32),
            jax.ShapeDtypeStruct((NWORK, EMB), jnp.float32),
            jax.ShapeDtypeStruct((NWORK, EMB), jnp.float32),
            jax.ShapeDtypeStruct((E,), jnp.int32),
        ),
        mesh=mesh,
        scratch_types=[
            pltpu.VMEM((CHUNK,), jnp.int32),
            pltpu.VMEM((CHUNK,), jnp.int32),
            pltpu.VMEM((CHUNK,), jnp.int32),
            pltpu.VMEM((CHUNK,), jnp.int32),
            pltpu.VMEM((CHUNK, 2 * EMB), jnp.float32),
            pltpu.VMEM((CHUNK, 2 * EMB), jnp.float32),
            pltpu.VMEM((CHUNK, EMB), jnp.float32),
            pltpu.VMEM((CHUNK, EMB), jnp.float32),
            pltpu.VMEM((CHUNK, EMB), jnp.float32),
            pltpu.VMEM((CHUNK, EMB), jnp.float32),
            pltpu.VMEM((CHUNK,), jnp.int32),
            pltpu.VMEM((2, EMB), jnp.float32),
            pltpu.VMEM_SHARED((NPAD, EMB), jnp.float32),
            pltpu.SemaphoreType.DMA,
            pltpu.SemaphoreType.DMA,
            pltpu.SemaphoreType.DMA,
            pltpu.SemaphoreType.DMA,
        ],
    )


def _row(x):
    return x.reshape(1, -1)


def _embed_h(h, pos_enc, sign, at_pad, pos_W, pos_b):
    return pl.pallas_call(
        _embed_h_body,
        grid=(N // NB,),
        in_specs=[
            pl.BlockSpec((NB, 9), lambda i: (i, 0)),
            pl.BlockSpec((NB, PE), lambda i: (i, 0)),
            pl.BlockSpec((1, PE), lambda i: (0, 0)),
            pl.BlockSpec((A_PAD, EMB), lambda i: (0, 0)),
            pl.BlockSpec((PE, EMB), lambda i: (0, 0)),
            pl.BlockSpec((1, EMB), lambda i: (0, 0)),
        ],
        out_specs=pl.BlockSpec((NB, EMB), lambda i: (i, 0)),
        out_shape=jax.ShapeDtypeStruct((N, EMB), jnp.float32),
    )(h, pos_enc, sign, at_pad, pos_W, pos_b)


def _embed_e(e_feat, bt_pad, wc, bc):
    return pl.pallas_call(
        _embed_e_body,
        grid=(E // EB,),
        in_specs=[
            pl.BlockSpec((EB, 3), lambda i: (i, 0)),
            pl.BlockSpec((B_PAD, EMB), lambda i: (0, 0)),
            pl.BlockSpec((EMB, EMB), lambda i: (0, 0)),
            pl.BlockSpec((1, EMB), lambda i: (0, 0)),
        ],
        out_specs=[
            pl.BlockSpec((EB, EMB), lambda i: (i, 0)),
            pl.BlockSpec((EB, EMB), lambda i: (i, 0)),
        ],
        out_shape=[
            jax.ShapeDtypeStruct((E, EMB), jnp.float32),
            jax.ShapeDtypeStruct((E, EMB), jnp.float32),
        ],
    )(e_feat, bt_pad, wc, bc)


def _nodemm(hx, batch2d, vn, wcat, bcat):
    return pl.pallas_call(
        _nodemm_body,
        grid=(N // NB,),
        in_specs=[
            pl.BlockSpec((NB, EMB), lambda i: (i, 0)),
            pl.BlockSpec((NB, 1), lambda i: (i, 0)),
            pl.BlockSpec((G, EMB), lambda i: (0, 0)),
            pl.BlockSpec((EMB, 4 * EMB), lambda i: (0, 0)),
            pl.BlockSpec((1, 4 * EMB), lambda i: (0, 0)),
        ],
        out_specs=[
            pl.BlockSpec((NB, EMB), lambda i: (i, 0)),
            pl.BlockSpec((NB, EMB), lambda i: (i, 0)),
            pl.BlockSpec((NB, 2 * EMB), lambda i: (i, 0)),
            pl.BlockSpec((NB, EMB), lambda i: (i, 0)),
        ],
        out_shape=[
            jax.ShapeDtypeStruct((N, EMB), jnp.float32),
            jax.ShapeDtypeStruct((N, EMB), jnp.float32),
            jax.ShapeDtypeStruct((N, 2 * EMB), jnp.float32),
            jax.ShapeDtypeStruct((N, EMB), jnp.float32),
        ],
    )(hx, batch2d, vn, wcat, bcat)


def _nodeup(heff, a, n0, n1, d0, d1, batch2d, g, b):
    hn, st = pl.pallas_call(
        _nodeup1_body,
        grid=(N // NB,),
        in_specs=[
            pl.BlockSpec((NB, EMB), lambda i: (i, 0)),
            pl.BlockSpec((NB, EMB), lambda i: (i, 0)),
            pl.BlockSpec((NB, EMB), lambda i: (i, 0)),
            pl.BlockSpec((NB, EMB), lambda i: (i, 0)),
            pl.BlockSpec((NB, EMB), lambda i: (i, 0)),
        ],
        out_specs=[
            pl.BlockSpec((NB, EMB), lambda i: (i, 0)),
            pl.BlockSpec((8, EMB), lambda i: (0, 0)),
        ],
        out_shape=[
            jax.ShapeDtypeStruct((N, EMB), jnp.float32),
            jax.ShapeDtypeStruct((8, EMB), jnp.float32),
        ],
    )(a, n0, n1, d0, d1)
    return pl.pallas_call(
        _nodeup2_body,
        grid=(N // NB,),
        in_specs=[
            pl.BlockSpec((NB, EMB), lambda i: (i, 0)),
            pl.BlockSpec((NB, EMB), lambda i: (i, 0)),
            pl.BlockSpec((8, EMB), lambda i: (0, 0)),
            pl.BlockSpec((1, EMB), lambda i: (0, 0)),
            pl.BlockSpec((1, EMB), lambda i: (0, 0)),
            pl.BlockSpec((NB, 1), lambda i: (i, 0)),
        ],
        out_specs=[
            pl.BlockSpec((NB, EMB), lambda i: (i, 0)),
            pl.BlockSpec((G, EMB), lambda i: (0, 0)),
            pl.BlockSpec((G, EMB), lambda i: (0, 0)),
        ],
        out_shape=[
            jax.ShapeDtypeStruct((N, EMB), jnp.float32),
            jax.ShapeDtypeStruct((G, EMB), jnp.float32),
            jax.ShapeDtypeStruct((G, EMB), jnp.float32),
        ],
    )(hn, heff, st, g, b, batch2d)


def _vn_mlp(vn, ps, pc, g, b, w1, b1, w2, b2):
    return pl.pallas_call(
        _vn_body,
        grid=(1,),
        in_specs=[
            pl.BlockSpec((G, EMB), lambda i: (0, 0)),
            pl.BlockSpec((G, EMB), lambda i: (0, 0)),
            pl.BlockSpec((G, EMB), lambda i: (0, 0)),
            pl.BlockSpec((1, EMB), lambda i: (0, 0)),
            pl.BlockSpec((1, EMB), lambda i: (0, 0)),
            pl.BlockSpec((EMB, HID), lambda i: (0, 0)),
            pl.BlockSpec((1, HID), lambda i: (0, 0)),
            pl.BlockSpec((HID, EMB), lambda i: (0, 0)),
            pl.BlockSpec((1, EMB), lambda i: (0, 0)),
        ],
        out_specs=pl.BlockSpec((G, EMB), lambda i: (0, 0)),
        out_shape=jax.ShapeDtypeStruct((G, EMB), jnp.float32),
    )(vn, ps, pc, g, b, w1, b1, w2, b2)


def _edgeup(enew, ex, esum, esq, g, b, wc, bc):
    return pl.pallas_call(
        _edgeup_body,
        grid=(E // EB,),
        in_specs=[
            pl.BlockSpec((EB, EMB), lambda i: (i, 0)),
            pl.BlockSpec((EB, EMB), lambda i: (i, 0)),
            pl.BlockSpec((NWORK, EMB), lambda i: (0, 0)),
            pl.BlockSpec((NWORK, EMB), lambda i: (0, 0)),
            pl.BlockSpec((1, EMB), lambda i: (0, 0)),
            pl.BlockSpec((1, EMB), lambda i: (0, 0)),
            pl.BlockSpec((EMB, EMB), lambda i: (0, 0)),
            pl.BlockSpec((1, EMB), lambda i: (0, 0)),
        ],
        out_specs=[
            pl.BlockSpec((EB, EMB), lambda i: (i, 0)),
            pl.BlockSpec((EB, EMB), lambda i: (i, 0)),
        ],
        out_shape=[
            jax.ShapeDtypeStruct((E, EMB), jnp.float32),
            jax.ShapeDtypeStruct((E, EMB), jnp.float32),
        ],
    )(enew, ex, esum, esq, g, b, wc, bc)


def _edgeup_final(enew, ex, esum, esq, g, b, ebatch2d):
    return pl.pallas_call(
        _edgeup_final_body,
        grid=(E // EB,),
        in_specs=[
            pl.BlockSpec((EB, EMB), lambda i: (i, 0)),
            pl.BlockSpec((EB, EMB), lambda i: (i, 0)),
            pl.BlockSpec((NWORK, EMB), lambda i: (0, 0)),
            pl.BlockSpec((NWORK, EMB), lambda i: (0, 0)),
            pl.BlockSpec((1, EMB), lambda i: (0, 0)),
            pl.BlockSpec((1, EMB), lambda i: (0, 0)),
            pl.BlockSpec((8, EB), lambda i: (0, i)),
        ],
        out_specs=[
            pl.BlockSpec((G, EMB), lambda i: (0, 0)),
            pl.BlockSpec((G, EMB), lambda i: (0, 0)),
        ],
        out_shape=[
            jax.ShapeDtypeStruct((G, EMB), jnp.float32),
            jax.ShapeDtypeStruct((G, EMB), jnp.float32),
        ],
    )(enew, ex, esum, esq, g, b, ebatch2d)


def _pred(nps, npc, esum, ecnt, w1, b1, w2, b2):
    return pl.pallas_call(
        _pred_body,
        grid=(1,),
        in_specs=[
            pl.BlockSpec((G, EMB), lambda i: (0, 0)),
            pl.BlockSpec((G, EMB), lambda i: (0, 0)),
            pl.BlockSpec((G, EMB), lambda i: (0, 0)),
            pl.BlockSpec((G, EMB), lambda i: (0, 0)),
            pl.BlockSpec((2 * EMB, HID), lambda i: (0, 0)),
            pl.BlockSpec((1, HID), lambda i: (0, 0)),
            pl.BlockSpec((HID, EMB), lambda i: (0, 0)),
            pl.BlockSpec((1, EMB), lambda i: (0, 0)),
        ],
        out_specs=pl.BlockSpec((G, TASKS), lambda i: (0, 0)),
        out_shape=jax.ShapeDtypeStruct((G, TASKS), jnp.float32),
    )(nps, npc, esum, ecnt, w1, b1, w2, b2)


def kernel(h, e_feat, edge_index, pos_enc, batch_index, atom_table,
           bond_table, pos_W, pos_b, layer_W, layer_b, bn_h_g, bn_h_b,
           bn_e_g, bn_e_b, vn_bn_g, vn_bn_b, vn_W1, vn_b1, vn_W2, vn_b2,
           pred_W1, pred_b1, pred_W2, pred_b2):
    sign = jnp.where(
        jax.random.randint(jax.random.key(42), (1, PE), 0, 2) == 0,
        -1.0, 1.0).astype(jnp.float32)
    at_pad = jnp.zeros((A_PAD, EMB), jnp.float32).at[:atom_table.shape[0]] \
        .set(atom_table)
    bt_pad = jnp.zeros((B_PAD, EMB), jnp.float32).at[:bond_table.shape[0]] \
        .set(bond_table)
    src = edge_index[0].astype(jnp.int32)
    dst = edge_index[1].astype(jnp.int32)
    batch1d = batch_index.astype(jnp.int32)
    batch2d = batch1d.reshape(N, 1)

    hx = _embed_h(h.astype(jnp.int32), pos_enc, sign, at_pad, pos_W,
                  _row(pos_b))
    ex, ce = _embed_e(e_feat.astype(jnp.int32), bt_pad, layer_W[0, 2],
                      _row(layer_b[0, 2]))

    edge_a = {False: _make_edge_a(False), True: _make_edge_a(True)}

    vn = jnp.zeros((G, EMB), jnp.float32)
    for l in range(L):
        wcat = jnp.concatenate(
            [layer_W[l, 0], layer_W[l, 1], layer_W[l, 3], layer_W[l, 4]], 1)
        bcat = jnp.concatenate(
            [layer_b[l, 0], layer_b[l, 1], layer_b[l, 3], layer_b[l, 4]], 0) \
            .reshape(1, 4 * EMB)
        heff, a, bd, eh = _nodemm(hx, batch2d, vn, wcat, bcat)
        last = l == L - 1
        enew, n0, n1, d0, d1, es, eq, eb_out = edge_a[last](
            bd, eh, ce, src, dst, batch1d)
        hx, nps, npc = _nodeup(heff, a, n0, n1, d0, d1,
                               batch2d, _row(bn_h_g[l]), _row(bn_h_b[l]))
        if not last:
            vn = _vn_mlp(vn, nps, npc, _row(vn_bn_g[l]), _row(vn_bn_b[l]),
                         vn_W1[l], _row(vn_b1[l]), vn_W2[l], _row(vn_b2[l]))
            ex, ce = _edgeup(enew, ex, es, eq, _row(bn_e_g[l]),
                             _row(bn_e_b[l]), layer_W[l + 1, 2],
                             _row(layer_b[l + 1, 2]))
        else:
            eps, epc = _edgeup_final(enew, ex, es, eq, _row(bn_e_g[l]),
                                     _row(bn_e_b[l]),
                                     jnp.broadcast_to(eb_out[None, :],
                                                      (8, E)))
    return _pred(nps, npc, eps, epc, pred_W1, _row(pred_b1), pred_W2,
                 _row(pred_b2))


# speedup vs baseline: 2.2917x; 1.0204x over previous
import functools

import jax
import jax.numpy as jnp
import numpy as np
from jax import lax
from jax.experimental import pallas as pl
from jax.experimental.pallas import tpu as pltpu
from jax.experimental.pallas import tpu_sc as plsc

N = 10000
E = 160000
EMB = 128
HID = 512
L = 5
G = 256
TASKS = 128
PE = 10
ATOM_DIMS = [119, 5, 12, 12, 10, 6, 6, 2, 2]
BOND_DIMS = [5, 6, 2]
ATOM_OFF = np.concatenate([[0], np.cumsum(ATOM_DIMS)[:-1]]).astype(np.int32)
BOND_OFF = np.concatenate([[0], np.cumsum(BOND_DIMS)[:-1]]).astype(np.int32)
A_PAD = 256
B_PAD = 16

NPAD = 10240
CHUNK = 40
NCHUNK = E // CHUNK
NWORK = 32
CPW = (NCHUNK + NWORK - 1) // NWORK
NSLOT = CPW + (CPW % 2)

NB = 2000
EB = 3200


def _embed_h_body(h_ref, pos_ref, sign_ref, at_ref, pw_ref, pb_ref, out_ref):
    hb = h_ref[...]
    acc = jnp.zeros((NB, A_PAD), jnp.float32)
    for i in range(9):
        col = hb[:, i:i + 1] + np.int32(ATOM_OFF[i])
        ids = lax.broadcasted_iota(jnp.int32, (NB, A_PAD), 1)
        acc = acc + (ids == col).astype(jnp.float32)
    hx = jnp.dot(acc, at_ref[...], preferred_element_type=jnp.float32)
    pos = pos_ref[...] * sign_ref[0:1, :]
    hx = hx + jnp.dot(pos, pw_ref[...], preferred_element_type=jnp.float32)
    out_ref[...] = hx + pb_ref[0:1, :]


def _embed_e_body(ef_ref, bt_ref, wc_ref, bc_ref, ex_ref, ce_ref):
    eb = ef_ref[...]
    acc = jnp.zeros((EB, B_PAD), jnp.float32)
    for i in range(3):
        col = eb[:, i:i + 1] + np.int32(BOND_OFF[i])
        ids = lax.broadcasted_iota(jnp.int32, (EB, B_PAD), 1)
        acc = acc + (ids == col).astype(jnp.float32)
    ex = jnp.dot(acc, bt_ref[...], preferred_element_type=jnp.float32)
    ex_ref[...] = ex
    ce_ref[...] = jnp.dot(ex, wc_ref[...], preferred_element_type=jnp.float32) \
        + bc_ref[0:1, :]


def _nodemm_body(hx_ref, b_ref, vn_ref, w_ref, bias_ref,
                 heff_ref, a_ref, bd_ref, e_ref):
    batch = b_ref[...]
    ids = lax.broadcasted_iota(jnp.int32, (NB, G), 1)
    mask = (ids == batch).astype(jnp.float32)
    heff = hx_ref[...] + jnp.dot(mask, vn_ref[...],
                                 preferred_element_type=jnp.float32)
    p = jnp.dot(heff, w_ref[...], preferred_element_type=jnp.float32) \
        + bias_ref[0:1, :]
    heff_ref[...] = heff
    a_ref[...] = p[:, 0:EMB]
    bd_ref[...] = p[:, EMB:3 * EMB]
    e_ref[...] = p[:, 3 * EMB:4 * EMB]


def _nodeup1_body(a_ref, n0_ref, n1_ref, d0_ref, d1_ref, hn_ref, st_ref):
    num = n0_ref[...] + n1_ref[...]
    den = d0_ref[...] + d1_ref[...] + 1e-6
    hnew = a_ref[...] + num / den
    hn_ref[...] = hnew
    s = jnp.sum(hnew, axis=0, keepdims=True)
    sq = jnp.sum(hnew * hnew, axis=0, keepdims=True)
    part = jnp.concatenate(
        [s, sq, jnp.zeros((6, EMB), jnp.float32)], axis=0)

    @pl.when(pl.program_id(0) == 0)
    def _init():
        st_ref[...] = jnp.zeros_like(st_ref)

    st_ref[...] += part


def _nodeup2_body(hn_ref, heff_ref, st_ref, g_ref, bb_ref, b_ref,
                  vn_ref, vg_ref, vb_ref, w1_ref, b1_ref, w2_ref, b2_ref,
                  hx_ref, ps_ref, pc_ref, vno_ref):
    s = st_ref[...]
    mu = s[0:1, :] / N
    var = s[1:2, :] / N - mu * mu
    hnew = (hn_ref[...] - mu) / jnp.sqrt(var + 1e-5) * g_ref[0:1, :] \
        + bb_ref[0:1, :]
    hx = heff_ref[...] + jnp.maximum(hnew, 0.0)
    hx_ref[...] = hx
    ids = lax.broadcasted_iota(jnp.int32, (NB, G), 1)
    maskn = (ids == b_ref[...]).astype(jnp.float32)
    dn = (((0,), (0,)), ((), ()))
    part = lax.dot_general(maskn, hx, dn,
                           preferred_element_type=jnp.float32)
    cntp = lax.dot_general(maskn, jnp.ones((NB, EMB), jnp.float32), dn,
                           preferred_element_type=jnp.float32)

    @pl.when(pl.program_id(0) == 0)
    def _init():
        ps_ref[...] = jnp.zeros_like(ps_ref)
        pc_ref[...] = jnp.zeros_like(pc_ref)

    ps_ref[...] += part
    pc_ref[...] += cntp

    @pl.when(pl.program_id(0) == pl.num_programs(0) - 1)
    def _vn():
        pool = ps_ref[...] / jnp.maximum(pc_ref[...], 1.0)
        vn = vn_ref[...] + pool
        mu = jnp.mean(vn, axis=0, keepdims=True)
        var = jnp.mean(vn * vn, axis=0, keepdims=True) - mu * mu
        t = (vn - mu) / jnp.sqrt(var + 1e-5) * vg_ref[0:1, :] + vb_ref[0:1, :]
        t = jnp.maximum(jnp.dot(t, w1_ref[...],
                                preferred_element_type=jnp.float32)
                        + b1_ref[0:1, :], 0.0)
        vno_ref[...] = jnp.dot(t, w2_ref[...],
                               preferred_element_type=jnp.float32) \
            + b2_ref[0:1, :]


def _ebn(en, esum_ref, esq_ref, g_ref, b_ref):
    mu = jnp.sum(esum_ref[...], axis=0, keepdims=True) / E
    var = jnp.sum(esq_ref[...], axis=0, keepdims=True) / E - mu * mu
    return jnp.maximum((en - mu) / jnp.sqrt(var + 1e-5) * g_ref[0:1, :]
                       + b_ref[0:1, :], 0.0)


def _edgeup_body(en_ref, ex_ref, esum_ref, esq_ref, g_ref, b_ref,
                 wc_ref, bc_ref, exo_ref, ce_ref):
    ex = ex_ref[...] + _ebn(en_ref[...], esum_ref, esq_ref, g_ref, b_ref)
    exo_ref[...] = ex
    ce_ref[...] = jnp.dot(ex, wc_ref[...],
                          preferred_element_type=jnp.float32) + bc_ref[0:1, :]


def _edgeup_final_body(en_ref, ex_ref, esum_ref, esq_ref, g_ref, b_ref,
                       eb_ref, ps_ref, pc_ref):
    ex = ex_ref[...] + _ebn(en_ref[...], esum_ref, esq_ref, g_ref, b_ref)
    bcol = jnp.broadcast_to(eb_ref[0:1, :], (G, EB))
    mask = (bcol == lax.broadcasted_iota(jnp.int32, (G, EB), 0)) \
        .astype(jnp.float32)
    part = jnp.dot(mask, ex, preferred_element_type=jnp.float32)
    cnt = jnp.sum(mask, axis=1, keepdims=True)

    @pl.when(pl.program_id(0) == 0)
    def _init():
        ps_ref[...] = jnp.zeros_like(ps_ref)
        pc_ref[...] = jnp.zeros_like(pc_ref)

    ps_ref[...] += part
    pc_ref[...] += jnp.broadcast_to(cnt, (G, EMB))


def _pred_body(nps_ref, npc_ref, es_ref, ec_ref, w1_ref, b1_ref, w2_ref,
               b2_ref, out_ref):
    node_pool = nps_ref[...] / jnp.maximum(npc_ref[...], 1.0)
    epool = es_ref[...] / jnp.maximum(ec_ref[...], 1.0)
    hg = jnp.concatenate([node_pool, epool], axis=-1)
    t = jnp.maximum(jnp.dot(hg, w1_ref[...],
                            preferred_element_type=jnp.float32)
                    + b1_ref[0:1, :], 0.0)
    out_ref[...] = jnp.dot(t, w2_ref[...],
                           preferred_element_type=jnp.float32) + b2_ref[0:1, :]


def _sc_zero_buf(buf):
    def zrow(r, _):
        for j in range(EMB // 16):
            buf[r, pl.ds(j * 16, 16)] = jnp.zeros((16,), jnp.float32)
        return 0
    lax.fori_loop(0, CHUNK, zrow, 0)


def _sc_zero_acc(acc, zbuf, sid):
    _sc_zero_buf(zbuf)
    for t in range(640 // CHUNK):
        pltpu.sync_copy(zbuf, acc.at[pl.ds(sid * 640 + t * CHUNK, CHUNK)])


def _sc_copy_out(acc, cid, sid, out0, out1):
    @pl.when(cid == 0)
    def _():
        pltpu.sync_copy(acc.at[pl.ds(sid * 640, 640)],
                        out0.at[pl.ds(sid * 640, 640)])

    @pl.when(cid == 1)
    def _():
        pltpu.sync_copy(acc.at[pl.ds(sid * 640, 640)],
                        out1.at[pl.ds(sid * 640, 640)])


def _edge_a_body(want_ebatch,
                 bd_hbm, eh_hbm, ce_hbm, src_hbm, dst_hbm, batch_hbm,
                 enew_hbm, num0_hbm, num1_hbm, den0_hbm, den1_hbm,
                 esum_hbm, esq_hbm, ebatch_hbm,
                 srcv0, srcv1, dstv0, dstv1, bdv0, bdv1, ev0, ev1,
                 cv0, cv1, btv, stats_v, acc,
                 semi0, semi1, semg0, semg1, semw0, semw1):
    cid = lax.axis_index("c")
    sid = lax.axis_index("s")
    wid = sid * 2 + cid
    srcv = (srcv0, srcv1)
    dstv = (dstv0, dstv1)
    bdv = (bdv0, bdv1)
    ev = (ev0, ev1)
    cv = (cv0, cv1)
    semi = (semi0, semi1)
    semg = (semg0, semg1)
    semw = (semw0, semw1)

    _sc_zero_acc(acc, cv0, sid)
    for r in range(2):
        for j in range(EMB // 16):
            stats_v[r, pl.ds(j * 16, 16)] = jnp.zeros((16,), jnp.float32)
    plsc.subcore_barrier()

    def _pred(i):
        return (wid + NWORK * i) < NCHUNK

    def _base(i):
        return (wid + NWORK * i) * CHUNK

    def _fire_idx(i, b, which):
        @pl.when(_pred(i))
        def _():
            if which in (0, 2):
                pltpu.async_copy(src_hbm.at[pl.ds(_base(i), CHUNK)],
                                 srcv[b], semi[b])
            if which in (1, 2):
                pltpu.async_copy(dst_hbm.at[pl.ds(_base(i), CHUNK)],
                                 dstv[b], semi[b])

    def _fire_gathers(i, b):
        @pl.when(jnp.logical_and(i >= 2, _pred(i - 2)))
        def _drain():
            pltpu.make_async_copy(cv[b], enew_hbm.at[pl.ds(0, CHUNK)],
                                  semw[b]).wait()

        @pl.when(_pred(i))
        def _():
            pltpu.make_async_copy(src_hbm.at[pl.ds(0, CHUNK)], srcv[b],
                                  semi[b]).wait()
            pltpu.make_async_copy(dst_hbm.at[pl.ds(0, CHUNK)], dstv[b],
                                  semi[b]).wait()
            pltpu.async_copy(bd_hbm.at[srcv[b]], bdv[b], semg[b])
            pltpu.async_copy(eh_hbm.at[dstv[b]], ev[b], semg[b])
            pltpu.async_copy(ce_hbm.at[pl.ds(_base(i), CHUNK)], cv[b],
                             semg[b])

    def _consume(i, b):
        @pl.when(_pred(i))
        def _():
            base = _base(i)
            pltpu.make_async_copy(bd_hbm.at[srcv[b]], bdv[b], semg[b]).wait()
            pltpu.make_async_copy(eh_hbm.at[dstv[b]], ev[b], semg[b]).wait()
            pltpu.make_async_copy(ce_hbm.at[pl.ds(base, CHUNK)], cv[b],
                                  semg[b]).wait()

            zeros16 = jnp.zeros((16,), jnp.float32)
            carry0 = (tuple(zeros16 for _ in range(EMB // 16)),
                      tuple(zeros16 for _ in range(EMB // 16)))

            @plsc.parallel_loop(0, CHUNK, unroll=4, carry=carry0)
            def _rows(r, cstat):
                s0, s1 = cstat
                n0, n1 = [], []
                for j in range(EMB // 16):
                    sl = pl.ds(j * 16, 16)
                    e = cv[b][r, sl] + bdv[b][r, pl.ds(EMB + j * 16, 16)] \
                        + ev[b][r, sl]
                    cv[b][r, sl] = e
                    sg = 1.0 / (1.0 + jnp.exp(-e))
                    ev[b][r, sl] = sg * bdv[b][r, sl]
                    n0.append(s0[j] + e)
                    n1.append(s1[j] + e * e)
                return (tuple(n0), tuple(n1))

            s0, s1 = _rows
            for j in range(EMB // 16):
                sl = pl.ds(j * 16, 16)
                stats_v[0, sl] = stats_v[0, sl] + s0[j]
                stats_v[1, sl] = stats_v[1, sl] + s1[j]
            pltpu.async_copy(cv[b], enew_hbm.at[pl.ds(base, CHUNK)], semw[b])
            pltpu.sync_copy(ev[b], acc.at[dstv[b]], add=True)
            if want_ebatch:
                pltpu.async_copy(batch_hbm.at[dstv[b]], btv, semg[b]).wait()
                pltpu.sync_copy(btv, ebatch_hbm.at[pl.ds(base, CHUNK)])

    _fire_idx(0, 0, 2)
    _fire_idx(1, 1, 2)
    _fire_gathers(0, 0)

    def slot_pair(io, _):
        for b in range(2):
            i = 2 * io + b
            _fire_gathers(i + 1, 1 - b)
            _consume(i, b)
            _fire_idx(i + 2, b, 0)
            _fire_idx(i + 2, b, 1)
        return 0

    lax.fori_loop(0, NSLOT // 2, slot_pair, 0)
    pltpu.sync_copy(stats_v.at[0], esum_hbm.at[wid])
    pltpu.sync_copy(stats_v.at[1], esq_hbm.at[wid])
    plsc.subcore_barrier()
    _sc_copy_out(acc, cid, sid, num0_hbm, num1_hbm)

    _sc_zero_acc(acc, cv0, sid)
    plsc.subcore_barrier()

    def _fire2(i, b):
        @pl.when(_pred(i))
        def _():
            base = _base(i)
            pltpu.async_copy(enew_hbm.at[pl.ds(base, CHUNK)], cv[b], semg[b])
            pltpu.async_copy(dst_hbm.at[pl.ds(base, CHUNK)], dstv[b], semg[b])

    def _consume2(i, b):
        @pl.when(_pred(i))
        def _():
            pltpu.make_async_copy(enew_hbm.at[pl.ds(0, CHUNK)], cv[b],
                                  semg[b]).wait()
            pltpu.make_async_copy(dst_hbm.at[pl.ds(0, CHUNK)], dstv[b],
                                  semg[b]).wait()

            @plsc.parallel_loop(0, CHUNK, unroll=4)
            def _rows2(r):
                for j in range(EMB // 16):
                    sl = pl.ds(j * 16, 16)
                    cv[b][r, sl] = 1.0 / (1.0 + jnp.exp(-cv[b][r, sl]))

            pltpu.sync_copy(cv[b], acc.at[dstv[b]], add=True)

    _fire2(0, 0)

    def slot_pair2(io, _):
        for b in range(2):
            i = 2 * io + b
            _fire2(i + 1, 1 - b)
            _consume2(i, b)
        return 0

    lax.fori_loop(0, NSLOT // 2, slot_pair2, 0)
    plsc.subcore_barrier()
    _sc_copy_out(acc, cid, sid, den0_hbm, den1_hbm)


def _make_edge_a(want_ebatch):
    mesh = plsc.VectorSubcoreMesh(core_axis_name="c", subcore_axis_name="s")
    return pl.kernel(
        functools.partial(_edge_a_body, want_ebatch),
        out_type=(
            jax.ShapeDtypeStruct((E, EMB), jnp.float32),
            jax.ShapeDtypeStruct((NPAD, EMB), jnp.float32),
            jax.ShapeDtypeStruct((NPAD, EMB), jnp.float32),
            jax.ShapeDtypeStruct((NPAD, EMB), jnp.float32),
            jax.ShapeDtypeStruct((NPAD, EMB), jnp.float32),
            jax.ShapeDtypeStruct((NWORK, EMB), jnp.float32),
            jax.ShapeDtypeStruct((NWORK, EMB), jnp.float32),
            jax.ShapeDtypeStruct((E,), jnp.int32),
        ),
        mesh=mesh,
        scratch_types=[
            pltpu.VMEM((CHUNK,), jnp.int32),
            pltpu.VMEM((CHUNK,), jnp.int32),
            pltpu.VMEM((CHUNK,), jnp.int32),
            pltpu.VMEM((CHUNK,), jnp.int32),
            pltpu.VMEM((CHUNK, 2 * EMB), jnp.float32),
            pltpu.VMEM((CHUNK, 2 * EMB), jnp.float32),
            pltpu.VMEM((CHUNK, EMB), jnp.float32),
            pltpu.VMEM((CHUNK, EMB), jnp.float32),
            pltpu.VMEM((CHUNK, EMB), jnp.float32),
            pltpu.VMEM((CHUNK, EMB), jnp.float32),
            pltpu.VMEM((CHUNK,), jnp.int32),
            pltpu.VMEM((2, EMB), jnp.float32),
            pltpu.VMEM_SHARED((NPAD, EMB), jnp.float32),
            pltpu.SemaphoreType.DMA,
            pltpu.SemaphoreType.DMA,
            pltpu.SemaphoreType.DMA,
            pltpu.SemaphoreType.DMA,
            pltpu.SemaphoreType.DMA,
            pltpu.SemaphoreType.DMA,
        ],
    )


def _row(x):
    return x.reshape(1, -1)


def _embed_h(h, pos_enc, sign, at_pad, pos_W, pos_b):
    return pl.pallas_call(
        _embed_h_body,
        grid=(N // NB,),
        in_specs=[
            pl.BlockSpec((NB, 9), lambda i: (i, 0)),
            pl.BlockSpec((NB, PE), lambda i: (i, 0)),
            pl.BlockSpec((1, PE), lambda i: (0, 0)),
            pl.BlockSpec((A_PAD, EMB), lambda i: (0, 0)),
            pl.BlockSpec((PE, EMB), lambda i: (0, 0)),
            pl.BlockSpec((1, EMB), lambda i: (0, 0)),
        ],
        out_specs=pl.BlockSpec((NB, EMB), lambda i: (i, 0)),
        out_shape=jax.ShapeDtypeStruct((N, EMB), jnp.float32),
    )(h, pos_enc, sign, at_pad, pos_W, pos_b)


def _embed_e(e_feat, bt_pad, wc, bc):
    return pl.pallas_call(
        _embed_e_body,
        grid=(E // EB,),
        in_specs=[
            pl.BlockSpec((EB, 3), lambda i: (i, 0)),
            pl.BlockSpec((B_PAD, EMB), lambda i: (0, 0)),
            pl.BlockSpec((EMB, EMB), lambda i: (0, 0)),
            pl.BlockSpec((1, EMB), lambda i: (0, 0)),
        ],
        out_specs=[
            pl.BlockSpec((EB, EMB), lambda i: (i, 0)),
            pl.BlockSpec((EB, EMB), lambda i: (i, 0)),
        ],
        out_shape=[
            jax.ShapeDtypeStruct((E, EMB), jnp.float32),
            jax.ShapeDtypeStruct((E, EMB), jnp.float32),
        ],
    )(e_feat, bt_pad, wc, bc)


def _nodemm(hx, batch2d, vn, wcat, bcat):
    return pl.pallas_call(
        _nodemm_body,
        grid=(N // NB,),
        in_specs=[
            pl.BlockSpec((NB, EMB), lambda i: (i, 0)),
            pl.BlockSpec((NB, 1), lambda i: (i, 0)),
            pl.BlockSpec((G, EMB), lambda i: (0, 0)),
            pl.BlockSpec((EMB, 4 * EMB), lambda i: (0, 0)),
            pl.BlockSpec((1, 4 * EMB), lambda i: (0, 0)),
        ],
        out_specs=[
            pl.BlockSpec((NB, EMB), lambda i: (i, 0)),
            pl.BlockSpec((NB, EMB), lambda i: (i, 0)),
            pl.BlockSpec((NB, 2 * EMB), lambda i: (i, 0)),
            pl.BlockSpec((NB, EMB), lambda i: (i, 0)),
        ],
        out_shape=[
            jax.ShapeDtypeStruct((N, EMB), jnp.float32),
            jax.ShapeDtypeStruct((N, EMB), jnp.float32),
            jax.ShapeDtypeStruct((N, 2 * EMB), jnp.float32),
            jax.ShapeDtypeStruct((N, EMB), jnp.float32),
        ],
    )(hx, batch2d, vn, wcat, bcat)


def _nodeup(heff, a, n0, n1, d0, d1, batch2d, g, b,
            vn, vg, vb, vw1, vb1, vw2, vb2):
    hn, st = pl.pallas_call(
        _nodeup1_body,
        grid=(N // NB,),
        in_specs=[
            pl.BlockSpec((NB, EMB), lambda i: (i, 0)),
            pl.BlockSpec((NB, EMB), lambda i: (i, 0)),
            pl.BlockSpec((NB, EMB), lambda i: (i, 0)),
            pl.BlockSpec((NB, EMB), lambda i: (i, 0)),
            pl.BlockSpec((NB, EMB), lambda i: (i, 0)),
        ],
        out_specs=[
            pl.BlockSpec((NB, EMB), lambda i: (i, 0)),
            pl.BlockSpec((8, EMB), lambda i: (0, 0)),
        ],
        out_shape=[
            jax.ShapeDtypeStruct((N, EMB), jnp.float32),
            jax.ShapeDtypeStruct((8, EMB), jnp.float32),
        ],
    )(a, n0, n1, d0, d1)
    return pl.pallas_call(
        _nodeup2_body,
        grid=(N // NB,),
        in_specs=[
            pl.BlockSpec((NB, EMB), lambda i: (i, 0)),
            pl.BlockSpec((NB, EMB), lambda i: (i, 0)),
            pl.BlockSpec((8, EMB), lambda i: (0, 0)),
            pl.BlockSpec((1, EMB), lambda i: (0, 0)),
            pl.BlockSpec((1, EMB), lambda i: (0, 0)),
            pl.BlockSpec((NB, 1), lambda i: (i, 0)),
            pl.BlockSpec((G, EMB), lambda i: (0, 0)),
            pl.BlockSpec((1, EMB), lambda i: (0, 0)),
            pl.BlockSpec((1, EMB), lambda i: (0, 0)),
            pl.BlockSpec((EMB, HID), lambda i: (0, 0)),
            pl.BlockSpec((1, HID), lambda i: (0, 0)),
            pl.BlockSpec((HID, EMB), lambda i: (0, 0)),
            pl.BlockSpec((1, EMB), lambda i: (0, 0)),
        ],
        out_specs=[
            pl.BlockSpec((NB, EMB), lambda i: (i, 0)),
            pl.BlockSpec((G, EMB), lambda i: (0, 0)),
            pl.BlockSpec((G, EMB), lambda i: (0, 0)),
            pl.BlockSpec((G, EMB), lambda i: (0, 0)),
        ],
        out_shape=[
            jax.ShapeDtypeStruct((N, EMB), jnp.float32),
            jax.ShapeDtypeStruct((G, EMB), jnp.float32),
            jax.ShapeDtypeStruct((G, EMB), jnp.float32),
            jax.ShapeDtypeStruct((G, EMB), jnp.float32),
        ],
    )(hn, heff, st, g, b, batch2d, vn, vg, vb, vw1, vb1, vw2, vb2)


def _edgeup(enew, ex, esum, esq, g, b, wc, bc):
    return pl.pallas_call(
        _edgeup_body,
        grid=(E // EB,),
        in_specs=[
            pl.BlockSpec((EB, EMB), lambda i: (i, 0)),
            pl.BlockSpec((EB, EMB), lambda i: (i, 0)),
            pl.BlockSpec((NWORK, EMB), lambda i: (0, 0)),
            pl.BlockSpec((NWORK, EMB), lambda i: (0, 0)),
            pl.BlockSpec((1, EMB), lambda i: (0, 0)),
            pl.BlockSpec((1, EMB), lambda i: (0, 0)),
            pl.BlockSpec((EMB, EMB), lambda i: (0, 0)),
            pl.BlockSpec((1, EMB), lambda i: (0, 0)),
        ],
        out_specs=[
            pl.BlockSpec((EB, EMB), lambda i: (i, 0)),
            pl.BlockSpec((EB, EMB), lambda i: (i, 0)),
        ],
        out_shape=[
            jax.ShapeDtypeStruct((E, EMB), jnp.float32),
            jax.ShapeDtypeStruct((E, EMB), jnp.float32),
        ],
    )(enew, ex, esum, esq, g, b, wc, bc)


def _edgeup_final(enew, ex, esum, esq, g, b, ebatch2d):
    return pl.pallas_call(
        _edgeup_final_body,
        grid=(E // EB,),
        in_specs=[
            pl.BlockSpec((EB, EMB), lambda i: (i, 0)),
            pl.BlockSpec((EB, EMB), lambda i: (i, 0)),
            pl.BlockSpec((NWORK, EMB), lambda i: (0, 0)),
            pl.BlockSpec((NWORK, EMB), lambda i: (0, 0)),
            pl.BlockSpec((1, EMB), lambda i: (0, 0)),
            pl.BlockSpec((1, EMB), lambda i: (0, 0)),
            pl.BlockSpec((8, EB), lambda i: (0, i)),
        ],
        out_specs=[
            pl.BlockSpec((G, EMB), lambda i: (0, 0)),
            pl.BlockSpec((G, EMB), lambda i: (0, 0)),
        ],
        out_shape=[
            jax.ShapeDtypeStruct((G, EMB), jnp.float32),
            jax.ShapeDtypeStruct((G, EMB), jnp.float32),
        ],
    )(enew, ex, esum, esq, g, b, ebatch2d)


def _pred(nps, npc, esum, ecnt, w1, b1, w2, b2):
    return pl.pallas_call(
        _pred_body,
        grid=(1,),
        in_specs=[
            pl.BlockSpec((G, EMB), lambda i: (0, 0)),
            pl.BlockSpec((G, EMB), lambda i: (0, 0)),
            pl.BlockSpec((G, EMB), lambda i: (0, 0)),
            pl.BlockSpec((G, EMB), lambda i: (0, 0)),
            pl.BlockSpec((2 * EMB, HID), lambda i: (0, 0)),
            pl.BlockSpec((1, HID), lambda i: (0, 0)),
            pl.BlockSpec((HID, EMB), lambda i: (0, 0)),
            pl.BlockSpec((1, EMB), lambda i: (0, 0)),
        ],
        out_specs=pl.BlockSpec((G, TASKS), lambda i: (0, 0)),
        out_shape=jax.ShapeDtypeStruct((G, TASKS), jnp.float32),
    )(nps, npc, esum, ecnt, w1, b1, w2, b2)


def kernel(h, e_feat, edge_index, pos_enc, batch_index, atom_table,
           bond_table, pos_W, pos_b, layer_W, layer_b, bn_h_g, bn_h_b,
           bn_e_g, bn_e_b, vn_bn_g, vn_bn_b, vn_W1, vn_b1, vn_W2, vn_b2,
           pred_W1, pred_b1, pred_W2, pred_b2):
    sign = jnp.where(
        jax.random.randint(jax.random.key(42), (1, PE), 0, 2) == 0,
        -1.0, 1.0).astype(jnp.float32)
    at_pad = jnp.zeros((A_PAD, EMB), jnp.float32).at[:atom_table.shape[0]] \
        .set(atom_table)
    bt_pad = jnp.zeros((B_PAD, EMB), jnp.float32).at[:bond_table.shape[0]] \
        .set(bond_table)
    src = edge_index[0].astype(jnp.int32)
    dst = edge_index[1].astype(jnp.int32)
    batch1d = batch_index.astype(jnp.int32)
    batch2d = batch1d.reshape(N, 1)

    hx = _embed_h(h.astype(jnp.int32), pos_enc, sign, at_pad, pos_W,
                  _row(pos_b))
    ex, ce = _embed_e(e_feat.astype(jnp.int32), bt_pad, layer_W[0, 2],
                      _row(layer_b[0, 2]))

    edge_a = {False: _make_edge_a(False), True: _make_edge_a(True)}

    vn = jnp.zeros((G, EMB), jnp.float32)
    for l in range(L):
        wcat = jnp.concatenate(
            [layer_W[l, 0], layer_W[l, 1], layer_W[l, 3], layer_W[l, 4]], 1)
        bcat = jnp.concatenate(
            [layer_b[l, 0], layer_b[l, 1], layer_b[l, 3], layer_b[l, 4]], 0) \
            .reshape(1, 4 * EMB)
        heff, a, bd, eh = _nodemm(hx, batch2d, vn, wcat, bcat)
        last = l == L - 1
        enew, n0, n1, d0, d1, es, eq, eb_out = edge_a[last](
            bd, eh, ce, src, dst, batch1d)
        lv = min(l, L - 2)
        hx, nps, npc, vn_new = _nodeup(
            heff, a, n0, n1, d0, d1, batch2d,
            _row(bn_h_g[l]), _row(bn_h_b[l]),
            vn, _row(vn_bn_g[lv]), _row(vn_bn_b[lv]),
            vn_W1[lv], _row(vn_b1[lv]), vn_W2[lv], _row(vn_b2[lv]))
        if not last:
            vn = vn_new
            ex, ce = _edgeup(enew, ex, es, eq, _row(bn_e_g[l]),
                             _row(bn_e_b[l]), layer_W[l + 1, 2],
                             _row(layer_b[l + 1, 2]))
        else:
            eps, epc = _edgeup_final(enew, ex, es, eq, _row(bn_e_g[l]),
                                     _row(bn_e_b[l]),
                                     jnp.broadcast_to(eb_out[None, :],
                                                      (8, E)))
    return _pred(nps, npc, eps, epc, pred_W1, _row(pred_b1), pred_W2,
                 _row(pred_b2))


# speedup vs baseline: 2.2927x; 1.0004x over previous
import functools

import jax
import jax.numpy as jnp
import numpy as np
from jax import lax
from jax.experimental import pallas as pl
from jax.experimental.pallas import tpu as pltpu
from jax.experimental.pallas import tpu_sc as plsc

N = 10000
E = 160000
EMB = 128
HID = 512
L = 5
G = 256
TASKS = 128
PE = 10
ATOM_DIMS = [119, 5, 12, 12, 10, 6, 6, 2, 2]
BOND_DIMS = [5, 6, 2]
ATOM_OFF = np.concatenate([[0], np.cumsum(ATOM_DIMS)[:-1]]).astype(np.int32)
BOND_OFF = np.concatenate([[0], np.cumsum(BOND_DIMS)[:-1]]).astype(np.int32)
A_PAD = 256
B_PAD = 16

NPAD = 10240
CHUNK = 40
NCHUNK = E // CHUNK
NWORK = 32
CPW = (NCHUNK + NWORK - 1) // NWORK
NSLOT = CPW + (CPW % 2)

NB = 2000
EB = 3200


def _embed_h_body(h_ref, pos_ref, sign_ref, at_ref, pw_ref, pb_ref, out_ref):
    hb = h_ref[...]
    acc = jnp.zeros((NB, A_PAD), jnp.float32)
    for i in range(9):
        col = hb[:, i:i + 1] + np.int32(ATOM_OFF[i])
        ids = lax.broadcasted_iota(jnp.int32, (NB, A_PAD), 1)
        acc = acc + (ids == col).astype(jnp.float32)
    hx = jnp.dot(acc, at_ref[...], preferred_element_type=jnp.float32)
    pos = pos_ref[...] * sign_ref[0:1, :]
    hx = hx + jnp.dot(pos, pw_ref[...], preferred_element_type=jnp.float32)
    out_ref[...] = hx + pb_ref[0:1, :]


def _embed_e_body(ef_ref, bt_ref, wc_ref, bc_ref, ex_ref, ce_ref):
    eb = ef_ref[...]
    acc = jnp.zeros((EB, B_PAD), jnp.float32)
    for i in range(3):
        col = eb[:, i:i + 1] + np.int32(BOND_OFF[i])
        ids = lax.broadcasted_iota(jnp.int32, (EB, B_PAD), 1)
        acc = acc + (ids == col).astype(jnp.float32)
    ex = jnp.dot(acc, bt_ref[...], preferred_element_type=jnp.float32)
    ex_ref[...] = ex
    ce_ref[...] = jnp.dot(ex, wc_ref[...], preferred_element_type=jnp.float32) \
        + bc_ref[0:1, :]


def _nodemm_body(hx_ref, b_ref, vn_ref, w_ref, bias_ref,
                 heff_ref, a_ref, bd_ref, e_ref):
    batch = b_ref[...]
    ids = lax.broadcasted_iota(jnp.int32, (NB, G), 1)
    mask = (ids == batch).astype(jnp.float32)
    heff = hx_ref[...] + jnp.dot(mask, vn_ref[...],
                                 preferred_element_type=jnp.float32)
    p = jnp.dot(heff, w_ref[...], preferred_element_type=jnp.float32) \
        + bias_ref[0:1, :]
    heff_ref[...] = heff
    a_ref[...] = p[:, 0:EMB]
    bd_ref[...] = p[:, EMB:3 * EMB]
    e_ref[...] = p[:, 3 * EMB:4 * EMB]


def _nodeup1_body(a_ref, n0_ref, n1_ref, d0_ref, d1_ref, hn_ref, st_ref):
    num = n0_ref[...] + n1_ref[...]
    den = d0_ref[...] + d1_ref[...] + 1e-6
    hnew = a_ref[...] + num / den
    hn_ref[...] = hnew
    s = jnp.sum(hnew, axis=0, keepdims=True)
    sq = jnp.sum(hnew * hnew, axis=0, keepdims=True)
    part = jnp.concatenate(
        [s, sq, jnp.zeros((6, EMB), jnp.float32)], axis=0)

    @pl.when(pl.program_id(0) == 0)
    def _init():
        st_ref[...] = jnp.zeros_like(st_ref)

    st_ref[...] += part


def _nodeup2_body(want_cnt, hn_ref, heff_ref, st_ref, g_ref, bb_ref, b_ref,
                  vn_ref, vg_ref, vb_ref, w1_ref, b1_ref, w2_ref, b2_ref,
                  *refs):
    if want_cnt:
        hx_ref, ps_ref, pc_ref, vno_ref = refs
    else:
        pcin_ref, hx_ref, ps_ref, vno_ref = refs
    s = st_ref[...]
    mu = s[0:1, :] / N
    var = s[1:2, :] / N - mu * mu
    hnew = (hn_ref[...] - mu) / jnp.sqrt(var + 1e-5) * g_ref[0:1, :] \
        + bb_ref[0:1, :]
    hx = heff_ref[...] + jnp.maximum(hnew, 0.0)
    hx_ref[...] = hx
    ids = lax.broadcasted_iota(jnp.int32, (NB, G), 1)
    maskn = (ids == b_ref[...]).astype(jnp.float32)
    dn = (((0,), (0,)), ((), ()))
    part = lax.dot_general(maskn, hx, dn,
                           preferred_element_type=jnp.float32)

    @pl.when(pl.program_id(0) == 0)
    def _init():
        ps_ref[...] = jnp.zeros_like(ps_ref)
        if want_cnt:
            pc_ref[...] = jnp.zeros_like(pc_ref)

    ps_ref[...] += part
    if want_cnt:
        cntp = lax.dot_general(maskn, jnp.ones((NB, EMB), jnp.float32), dn,
                               preferred_element_type=jnp.float32)
        pc_ref[...] += cntp

    @pl.when(pl.program_id(0) == pl.num_programs(0) - 1)
    def _vn():
        cnt = pc_ref[...] if want_cnt else pcin_ref[...]
        pool = ps_ref[...] / jnp.maximum(cnt, 1.0)
        vn = vn_ref[...] + pool
        mu = jnp.mean(vn, axis=0, keepdims=True)
        var = jnp.mean(vn * vn, axis=0, keepdims=True) - mu * mu
        t = (vn - mu) / jnp.sqrt(var + 1e-5) * vg_ref[0:1, :] + vb_ref[0:1, :]
        t = jnp.maximum(jnp.dot(t, w1_ref[...],
                                preferred_element_type=jnp.float32)
                        + b1_ref[0:1, :], 0.0)
        vno_ref[...] = jnp.dot(t, w2_ref[...],
                               preferred_element_type=jnp.float32) \
            + b2_ref[0:1, :]


def _ebn(en, esum_ref, esq_ref, g_ref, b_ref):
    mu = jnp.sum(esum_ref[...], axis=0, keepdims=True) / E
    var = jnp.sum(esq_ref[...], axis=0, keepdims=True) / E - mu * mu
    return jnp.maximum((en - mu) / jnp.sqrt(var + 1e-5) * g_ref[0:1, :]
                       + b_ref[0:1, :], 0.0)


def _edgeup_body(en_ref, ex_ref, esum_ref, esq_ref, g_ref, b_ref,
                 wc_ref, bc_ref, exo_ref, ce_ref):
    ex = ex_ref[...] + _ebn(en_ref[...], esum_ref, esq_ref, g_ref, b_ref)
    exo_ref[...] = ex
    ce_ref[...] = jnp.dot(ex, wc_ref[...],
                          preferred_element_type=jnp.float32) + bc_ref[0:1, :]


def _edgeup_final_body(en_ref, ex_ref, esum_ref, esq_ref, g_ref, b_ref,
                       eb_ref, ps_ref, pc_ref):
    ex = ex_ref[...] + _ebn(en_ref[...], esum_ref, esq_ref, g_ref, b_ref)
    bcol = jnp.broadcast_to(eb_ref[0:1, :], (G, EB))
    mask = (bcol == lax.broadcasted_iota(jnp.int32, (G, EB), 0)) \
        .astype(jnp.float32)
    part = jnp.dot(mask, ex, preferred_element_type=jnp.float32)
    cnt = jnp.sum(mask, axis=1, keepdims=True)

    @pl.when(pl.program_id(0) == 0)
    def _init():
        ps_ref[...] = jnp.zeros_like(ps_ref)
        pc_ref[...] = jnp.zeros_like(pc_ref)

    ps_ref[...] += part
    pc_ref[...] += jnp.broadcast_to(cnt, (G, EMB))


def _pred_body(nps_ref, npc_ref, es_ref, ec_ref, w1_ref, b1_ref, w2_ref,
               b2_ref, out_ref):
    node_pool = nps_ref[...] / jnp.maximum(npc_ref[...], 1.0)
    epool = es_ref[...] / jnp.maximum(ec_ref[...], 1.0)
    hg = jnp.concatenate([node_pool, epool], axis=-1)
    t = jnp.maximum(jnp.dot(hg, w1_ref[...],
                            preferred_element_type=jnp.float32)
                    + b1_ref[0:1, :], 0.0)
    out_ref[...] = jnp.dot(t, w2_ref[...],
                           preferred_element_type=jnp.float32) + b2_ref[0:1, :]


def _sc_zero_buf(buf):
    def zrow(r, _):
        for j in range(EMB // 16):
            buf[r, pl.ds(j * 16, 16)] = jnp.zeros((16,), jnp.float32)
        return 0
    lax.fori_loop(0, CHUNK, zrow, 0)


def _sc_zero_acc(acc, zbuf, sid):
    _sc_zero_buf(zbuf)
    for t in range(640 // CHUNK):
        pltpu.sync_copy(zbuf, acc.at[pl.ds(sid * 640 + t * CHUNK, CHUNK)])


def _sc_copy_out(acc, cid, sid, out0, out1):
    @pl.when(cid == 0)
    def _():
        pltpu.sync_copy(acc.at[pl.ds(sid * 640, 640)],
                        out0.at[pl.ds(sid * 640, 640)])

    @pl.when(cid == 1)
    def _():
        pltpu.sync_copy(acc.at[pl.ds(sid * 640, 640)],
                        out1.at[pl.ds(sid * 640, 640)])


def _edge_a_body(want_ebatch,
                 bd_hbm, eh_hbm, ce_hbm, src_hbm, dst_hbm, batch_hbm,
                 enew_hbm, num0_hbm, num1_hbm, den0_hbm, den1_hbm,
                 esum_hbm, esq_hbm, ebatch_hbm,
                 srcv0, srcv1, dstv0, dstv1, bdv0, bdv1, ev0, ev1,
                 cv0, cv1, btv, stats_v, acc,
                 semi0, semi1, semg0, semg1, semw0, semw1):
    cid = lax.axis_index("c")
    sid = lax.axis_index("s")
    wid = sid * 2 + cid
    srcv = (srcv0, srcv1)
    dstv = (dstv0, dstv1)
    bdv = (bdv0, bdv1)
    ev = (ev0, ev1)
    cv = (cv0, cv1)
    semi = (semi0, semi1)
    semg = (semg0, semg1)
    semw = (semw0, semw1)

    _sc_zero_acc(acc, cv0, sid)
    for r in range(2):
        for j in range(EMB // 16):
            stats_v[r, pl.ds(j * 16, 16)] = jnp.zeros((16,), jnp.float32)
    plsc.subcore_barrier()

    def _pred(i):
        return (wid + NWORK * i) < NCHUNK

    def _base(i):
        return (wid + NWORK * i) * CHUNK

    def _fire_idx(i, b, which):
        @pl.when(_pred(i))
        def _():
            if which in (0, 2):
                pltpu.async_copy(src_hbm.at[pl.ds(_base(i), CHUNK)],
                                 srcv[b], semi[b])
            if which in (1, 2):
                pltpu.async_copy(dst_hbm.at[pl.ds(_base(i), CHUNK)],
                                 dstv[b], semi[b])

    def _fire_gathers(i, b):
        @pl.when(jnp.logical_and(i >= 2, _pred(i - 2)))
        def _drain():
            pltpu.make_async_copy(cv[b], enew_hbm.at[pl.ds(0, CHUNK)],
                                  semw[b]).wait()

        @pl.when(_pred(i))
        def _():
            pltpu.make_async_copy(src_hbm.at[pl.ds(0, CHUNK)], srcv[b],
                                  semi[b]).wait()
            pltpu.make_async_copy(dst_hbm.at[pl.ds(0, CHUNK)], dstv[b],
                                  semi[b]).wait()
            pltpu.async_copy(bd_hbm.at[srcv[b]], bdv[b], semg[b])
            pltpu.async_copy(eh_hbm.at[dstv[b]], ev[b], semg[b])
            pltpu.async_copy(ce_hbm.at[pl.ds(_base(i), CHUNK)], cv[b],
                             semg[b])

    def _consume(i, b):
        @pl.when(_pred(i))
        def _():
            base = _base(i)
            pltpu.make_async_copy(bd_hbm.at[srcv[b]], bdv[b], semg[b]).wait()
            pltpu.make_async_copy(eh_hbm.at[dstv[b]], ev[b], semg[b]).wait()
            pltpu.make_async_copy(ce_hbm.at[pl.ds(base, CHUNK)], cv[b],
                                  semg[b]).wait()

            zeros16 = jnp.zeros((16,), jnp.float32)
            carry0 = (tuple(zeros16 for _ in range(EMB // 16)),
                      tuple(zeros16 for _ in range(EMB // 16)))

            @plsc.parallel_loop(0, CHUNK, unroll=4, carry=carry0)
            def _rows(r, cstat):
                s0, s1 = cstat
                n0, n1 = [], []
                for j in range(EMB // 16):
                    sl = pl.ds(j * 16, 16)
                    e = cv[b][r, sl] + bdv[b][r, pl.ds(EMB + j * 16, 16)] \
                        + ev[b][r, sl]
                    cv[b][r, sl] = e
                    sg = 1.0 / (1.0 + jnp.exp(-e))
                    ev[b][r, sl] = sg * bdv[b][r, sl]
                    n0.append(s0[j] + e)
                    n1.append(s1[j] + e * e)
                return (tuple(n0), tuple(n1))

            s0, s1 = _rows
            for j in range(EMB // 16):
                sl = pl.ds(j * 16, 16)
                stats_v[0, sl] = stats_v[0, sl] + s0[j]
                stats_v[1, sl] = stats_v[1, sl] + s1[j]
            pltpu.async_copy(cv[b], enew_hbm.at[pl.ds(base, CHUNK)], semw[b])
            pltpu.sync_copy(ev[b], acc.at[dstv[b]], add=True)
            if want_ebatch:
                pltpu.async_copy(batch_hbm.at[dstv[b]], btv, semg[b]).wait()
                pltpu.sync_copy(btv, ebatch_hbm.at[pl.ds(base, CHUNK)])

    _fire_idx(0, 0, 2)
    _fire_idx(1, 1, 2)
    _fire_gathers(0, 0)

    def slot_pair(io, _):
        for b in range(2):
            i = 2 * io + b
            _fire_gathers(i + 1, 1 - b)
            _consume(i, b)
            _fire_idx(i + 2, b, 0)
            _fire_idx(i + 2, b, 1)
        return 0

    lax.fori_loop(0, NSLOT // 2, slot_pair, 0)
    pltpu.sync_copy(stats_v.at[0], esum_hbm.at[wid])
    pltpu.sync_copy(stats_v.at[1], esq_hbm.at[wid])
    plsc.subcore_barrier()
    _sc_copy_out(acc, cid, sid, num0_hbm, num1_hbm)

    _sc_zero_acc(acc, cv0, sid)
    plsc.subcore_barrier()

    def _fire2(i, b):
        @pl.when(_pred(i))
        def _():
            base = _base(i)
            pltpu.async_copy(enew_hbm.at[pl.ds(base, CHUNK)], cv[b], semg[b])
            pltpu.async_copy(dst_hbm.at[pl.ds(base, CHUNK)], dstv[b], semg[b])

    def _consume2(i, b):
        @pl.when(_pred(i))
        def _():
            pltpu.make_async_copy(enew_hbm.at[pl.ds(0, CHUNK)], cv[b],
                                  semg[b]).wait()
            pltpu.make_async_copy(dst_hbm.at[pl.ds(0, CHUNK)], dstv[b],
                                  semg[b]).wait()

            @plsc.parallel_loop(0, CHUNK, unroll=4)
            def _rows2(r):
                for j in range(EMB // 16):
                    sl = pl.ds(j * 16, 16)
                    cv[b][r, sl] = 1.0 / (1.0 + jnp.exp(-cv[b][r, sl]))

            pltpu.sync_copy(cv[b], acc.at[dstv[b]], add=True)

    _fire2(0, 0)

    def slot_pair2(io, _):
        for b in range(2):
            i = 2 * io + b
            _fire2(i + 1, 1 - b)
            _consume2(i, b)
        return 0

    lax.fori_loop(0, NSLOT // 2, slot_pair2, 0)
    plsc.subcore_barrier()
    _sc_copy_out(acc, cid, sid, den0_hbm, den1_hbm)


def _make_edge_a(want_ebatch):
    mesh = plsc.VectorSubcoreMesh(core_axis_name="c", subcore_axis_name="s")
    return pl.kernel(
        functools.partial(_edge_a_body, want_ebatch),
        out_type=(
            jax.ShapeDtypeStruct((E, EMB), jnp.float32),
            jax.ShapeDtypeStruct((NPAD, EMB), jnp.float32),
            jax.ShapeDtypeStruct((NPAD, EMB), jnp.float32),
            jax.ShapeDtypeStruct((NPAD, EMB), jnp.float32),
            jax.ShapeDtypeStruct((NPAD, EMB), jnp.float32),
            jax.ShapeDtypeStruct((NWORK, EMB), jnp.float32),
            jax.ShapeDtypeStruct((NWORK, EMB), jnp.float32),
            jax.ShapeDtypeStruct((E,), jnp.int32),
        ),
        mesh=mesh,
        scratch_types=[
            pltpu.VMEM((CHUNK,), jnp.int32),
            pltpu.VMEM((CHUNK,), jnp.int32),
            pltpu.VMEM((CHUNK,), jnp.int32),
            pltpu.VMEM((CHUNK,), jnp.int32),
            pltpu.VMEM((CHUNK, 2 * EMB), jnp.float32),
            pltpu.VMEM((CHUNK, 2 * EMB), jnp.float32),
            pltpu.VMEM((CHUNK, EMB), jnp.float32),
            pltpu.VMEM((CHUNK, EMB), jnp.float32),
            pltpu.VMEM((CHUNK, EMB), jnp.float32),
            pltpu.VMEM((CHUNK, EMB), jnp.float32),
            pltpu.VMEM((CHUNK,), jnp.int32),
            pltpu.VMEM((2, EMB), jnp.float32),
            pltpu.VMEM_SHARED((NPAD, EMB), jnp.float32),
            pltpu.SemaphoreType.DMA,
            pltpu.SemaphoreType.DMA,
            pltpu.SemaphoreType.DMA,
            pltpu.SemaphoreType.DMA,
            pltpu.SemaphoreType.DMA,
            pltpu.SemaphoreType.DMA,
        ],
    )


def _row(x):
    return x.reshape(1, -1)


def _embed_h(h, pos_enc, sign, at_pad, pos_W, pos_b):
    return pl.pallas_call(
        _embed_h_body,
        grid=(N // NB,),
        in_specs=[
            pl.BlockSpec((NB, 9), lambda i: (i, 0)),
            pl.BlockSpec((NB, PE), lambda i: (i, 0)),
            pl.BlockSpec((1, PE), lambda i: (0, 0)),
            pl.BlockSpec((A_PAD, EMB), lambda i: (0, 0)),
            pl.BlockSpec((PE, EMB), lambda i: (0, 0)),
            pl.BlockSpec((1, EMB), lambda i: (0, 0)),
        ],
        out_specs=pl.BlockSpec((NB, EMB), lambda i: (i, 0)),
        out_shape=jax.ShapeDtypeStruct((N, EMB), jnp.float32),
    )(h, pos_enc, sign, at_pad, pos_W, pos_b)


def _embed_e(e_feat, bt_pad, wc, bc):
    return pl.pallas_call(
        _embed_e_body,
        grid=(E // EB,),
        in_specs=[
            pl.BlockSpec((EB, 3), lambda i: (i, 0)),
            pl.BlockSpec((B_PAD, EMB), lambda i: (0, 0)),
            pl.BlockSpec((EMB, EMB), lambda i: (0, 0)),
            pl.BlockSpec((1, EMB), lambda i: (0, 0)),
        ],
        out_specs=[
            pl.BlockSpec((EB, EMB), lambda i: (i, 0)),
            pl.BlockSpec((EB, EMB), lambda i: (i, 0)),
        ],
        out_shape=[
            jax.ShapeDtypeStruct((E, EMB), jnp.float32),
            jax.ShapeDtypeStruct((E, EMB), jnp.float32),
        ],
    )(e_feat, bt_pad, wc, bc)


def _nodemm(hx, batch2d, vn, wcat, bcat):
    return pl.pallas_call(
        _nodemm_body,
        grid=(N // NB,),
        in_specs=[
            pl.BlockSpec((NB, EMB), lambda i: (i, 0)),
            pl.BlockSpec((NB, 1), lambda i: (i, 0)),
            pl.BlockSpec((G, EMB), lambda i: (0, 0)),
            pl.BlockSpec((EMB, 4 * EMB), lambda i: (0, 0)),
            pl.BlockSpec((1, 4 * EMB), lambda i: (0, 0)),
        ],
        out_specs=[
            pl.BlockSpec((NB, EMB), lambda i: (i, 0)),
            pl.BlockSpec((NB, EMB), lambda i: (i, 0)),
            pl.BlockSpec((NB, 2 * EMB), lambda i: (i, 0)),
            pl.BlockSpec((NB, EMB), lambda i: (i, 0)),
        ],
        out_shape=[
            jax.ShapeDtypeStruct((N, EMB), jnp.float32),
            jax.ShapeDtypeStruct((N, EMB), jnp.float32),
            jax.ShapeDtypeStruct((N, 2 * EMB), jnp.float32),
            jax.ShapeDtypeStruct((N, EMB), jnp.float32),
        ],
    )(hx, batch2d, vn, wcat, bcat)


def _nodeup(heff, a, n0, n1, d0, d1, batch2d, g, b,
            vn, vg, vb, vw1, vb1, vw2, vb2, pc_prev=None):
    hn, st = pl.pallas_call(
        _nodeup1_body,
        grid=(N // NB,),
        in_specs=[
            pl.BlockSpec((NB, EMB), lambda i: (i, 0)),
            pl.BlockSpec((NB, EMB), lambda i: (i, 0)),
            pl.BlockSpec((NB, EMB), lambda i: (i, 0)),
            pl.BlockSpec((NB, EMB), lambda i: (i, 0)),
            pl.BlockSpec((NB, EMB), lambda i: (i, 0)),
        ],
        out_specs=[
            pl.BlockSpec((NB, EMB), lambda i: (i, 0)),
            pl.BlockSpec((8, EMB), lambda i: (0, 0)),
        ],
        out_shape=[
            jax.ShapeDtypeStruct((N, EMB), jnp.float32),
            jax.ShapeDtypeStruct((8, EMB), jnp.float32),
        ],
    )(a, n0, n1, d0, d1)
    want_cnt = pc_prev is None
    common_in = [
        pl.BlockSpec((NB, EMB), lambda i: (i, 0)),
        pl.BlockSpec((NB, EMB), lambda i: (i, 0)),
        pl.BlockSpec((8, EMB), lambda i: (0, 0)),
        pl.BlockSpec((1, EMB), lambda i: (0, 0)),
        pl.BlockSpec((1, EMB), lambda i: (0, 0)),
        pl.BlockSpec((NB, 1), lambda i: (i, 0)),
        pl.BlockSpec((G, EMB), lambda i: (0, 0)),
        pl.BlockSpec((1, EMB), lambda i: (0, 0)),
        pl.BlockSpec((1, EMB), lambda i: (0, 0)),
        pl.BlockSpec((EMB, HID), lambda i: (0, 0)),
        pl.BlockSpec((1, HID), lambda i: (0, 0)),
        pl.BlockSpec((HID, EMB), lambda i: (0, 0)),
        pl.BlockSpec((1, EMB), lambda i: (0, 0)),
    ]
    gspec = pl.BlockSpec((G, EMB), lambda i: (0, 0))
    args = [hn, heff, st, g, b, batch2d, vn, vg, vb, vw1, vb1, vw2, vb2]
    if want_cnt:
        hx, ps, pc, vno = pl.pallas_call(
            functools.partial(_nodeup2_body, True),
            grid=(N // NB,),
            in_specs=common_in,
            out_specs=[pl.BlockSpec((NB, EMB), lambda i: (i, 0)),
                       gspec, gspec, gspec],
            out_shape=[jax.ShapeDtypeStruct((N, EMB), jnp.float32)] +
                      [jax.ShapeDtypeStruct((G, EMB), jnp.float32)] * 3,
        )(*args)
        return hx, ps, pc, vno
    hx, ps, vno = pl.pallas_call(
        functools.partial(_nodeup2_body, False),
        grid=(N // NB,),
        in_specs=common_in + [gspec],
        out_specs=[pl.BlockSpec((NB, EMB), lambda i: (i, 0)), gspec, gspec],
        out_shape=[jax.ShapeDtypeStruct((N, EMB), jnp.float32)] +
                  [jax.ShapeDtypeStruct((G, EMB), jnp.float32)] * 2,
    )(*args, pc_prev)
    return hx, ps, pc_prev, vno


def _edgeup(enew, ex, esum, esq, g, b, wc, bc):
    return pl.pallas_call(
        _edgeup_body,
        grid=(E // EB,),
        in_specs=[
            pl.BlockSpec((EB, EMB), lambda i: (i, 0)),
            pl.BlockSpec((EB, EMB), lambda i: (i, 0)),
            pl.BlockSpec((NWORK, EMB), lambda i: (0, 0)),
            pl.BlockSpec((NWORK, EMB), lambda i: (0, 0)),
            pl.BlockSpec((1, EMB), lambda i: (0, 0)),
            pl.BlockSpec((1, EMB), lambda i: (0, 0)),
            pl.BlockSpec((EMB, EMB), lambda i: (0, 0)),
            pl.BlockSpec((1, EMB), lambda i: (0, 0)),
        ],
        out_specs=[
            pl.BlockSpec((EB, EMB), lambda i: (i, 0)),
            pl.BlockSpec((EB, EMB), lambda i: (i, 0)),
        ],
        out_shape=[
            jax.ShapeDtypeStruct((E, EMB), jnp.float32),
            jax.ShapeDtypeStruct((E, EMB), jnp.float32),
        ],
    )(enew, ex, esum, esq, g, b, wc, bc)


def _edgeup_final(enew, ex, esum, esq, g, b, ebatch2d):
    return pl.pallas_call(
        _edgeup_final_body,
        grid=(E // EB,),
        in_specs=[
            pl.BlockSpec((EB, EMB), lambda i: (i, 0)),
            pl.BlockSpec((EB, EMB), lambda i: (i, 0)),
            pl.BlockSpec((NWORK, EMB), lambda i: (0, 0)),
            pl.BlockSpec((NWORK, EMB), lambda i: (0, 0)),
            pl.BlockSpec((1, EMB), lambda i: (0, 0)),
            pl.BlockSpec((1, EMB), lambda i: (0, 0)),
            pl.BlockSpec((8, EB), lambda i: (0, i)),
        ],
        out_specs=[
            pl.BlockSpec((G, EMB), lambda i: (0, 0)),
            pl.BlockSpec((G, EMB), lambda i: (0, 0)),
        ],
        out_shape=[
            jax.ShapeDtypeStruct((G, EMB), jnp.float32),
            jax.ShapeDtypeStruct((G, EMB), jnp.float32),
        ],
    )(enew, ex, esum, esq, g, b, ebatch2d)


def _pred(nps, npc, esum, ecnt, w1, b1, w2, b2):
    return pl.pallas_call(
        _pred_body,
        grid=(1,),
        in_specs=[
            pl.BlockSpec((G, EMB), lambda i: (0, 0)),
            pl.BlockSpec((G, EMB), lambda i: (0, 0)),
            pl.BlockSpec((G, EMB), lambda i: (0, 0)),
            pl.BlockSpec((G, EMB), lambda i: (0, 0)),
            pl.BlockSpec((2 * EMB, HID), lambda i: (0, 0)),
            pl.BlockSpec((1, HID), lambda i: (0, 0)),
            pl.BlockSpec((HID, EMB), lambda i: (0, 0)),
            pl.BlockSpec((1, EMB), lambda i: (0, 0)),
        ],
        out_specs=pl.BlockSpec((G, TASKS), lambda i: (0, 0)),
        out_shape=jax.ShapeDtypeStruct((G, TASKS), jnp.float32),
    )(nps, npc, esum, ecnt, w1, b1, w2, b2)


def kernel(h, e_feat, edge_index, pos_enc, batch_index, atom_table,
           bond_table, pos_W, pos_b, layer_W, layer_b, bn_h_g, bn_h_b,
           bn_e_g, bn_e_b, vn_bn_g, vn_bn_b, vn_W1, vn_b1, vn_W2, vn_b2,
           pred_W1, pred_b1, pred_W2, pred_b2):
    sign = jnp.where(
        jax.random.randint(jax.random.key(42), (1, PE), 0, 2) == 0,
        -1.0, 1.0).astype(jnp.float32)
    at_pad = jnp.zeros((A_PAD, EMB), jnp.float32).at[:atom_table.shape[0]] \
        .set(atom_table)
    bt_pad = jnp.zeros((B_PAD, EMB), jnp.float32).at[:bond_table.shape[0]] \
        .set(bond_table)
    src = edge_index[0].astype(jnp.int32)
    dst = edge_index[1].astype(jnp.int32)
    batch1d = batch_index.astype(jnp.int32)
    batch2d = batch1d.reshape(N, 1)

    hx = _embed_h(h.astype(jnp.int32), pos_enc, sign, at_pad, pos_W,
                  _row(pos_b))
    ex, ce = _embed_e(e_feat.astype(jnp.int32), bt_pad, layer_W[0, 2],
                      _row(layer_b[0, 2]))

    edge_a = {False: _make_edge_a(False), True: _make_edge_a(True)}

    vn = jnp.zeros((G, EMB), jnp.float32)
    for l in range(L):
        wcat = jnp.concatenate(
            [layer_W[l, 0], layer_W[l, 1], layer_W[l, 3], layer_W[l, 4]], 1)
        bcat = jnp.concatenate(
            [layer_b[l, 0], layer_b[l, 1], layer_b[l, 3], layer_b[l, 4]], 0) \
            .reshape(1, 4 * EMB)
        heff, a, bd, eh = _nodemm(hx, batch2d, vn, wcat, bcat)
        last = l == L - 1
        enew, n0, n1, d0, d1, es, eq, eb_out = edge_a[last](
            bd, eh, ce, src, dst, batch1d)
        lv = min(l, L - 2)
        hx, nps, npc, vn_new = _nodeup(
            heff, a, n0, n1, d0, d1, batch2d,
            _row(bn_h_g[l]), _row(bn_h_b[l]),
            vn, _row(vn_bn_g[lv]), _row(vn_bn_b[lv]),
            vn_W1[lv], _row(vn_b1[lv]), vn_W2[lv], _row(vn_b2[lv]),
            pc_prev=None if l == 0 else npc)
        if not last:
            vn = vn_new
            ex, ce = _edgeup(enew, ex, es, eq, _row(bn_e_g[l]),
                             _row(bn_e_b[l]), layer_W[l + 1, 2],
                             _row(layer_b[l + 1, 2]))
        else:
            eps, epc = _edgeup_final(enew, ex, es, eq, _row(bn_e_g[l]),
                                     _row(bn_e_b[l]),
                                     jnp.broadcast_to(eb_out[None, :],
                                                      (8, E)))
    return _pred(nps, npc, eps, epc, pred_W1, _row(pred_b1), pred_W2,
                 _row(pred_b2))
